# Initial kernel scaffold; baseline (speedup 1.0000x reference)
import jax
import jax.numpy as jnp
from jax.experimental import pallas as pl


def kernel(xyz, points, t_embed, tw0, tb0, cw0, cb0, g0, b0, tw1, tb1, cw1, cb1, g1, b1, tw2, tb2, cw2, cb2, g2, b2):
    raise NotImplementedError("write your pallas kernel here")


# speedup vs baseline: 8.1403x; 8.1403x over previous
import functools

import jax
import jax.numpy as jnp
import numpy as np
from jax import lax
from jax.experimental import pallas as pl
from jax.experimental.pallas import tpu as pltpu

BN = 2
NN = 2048
KS = 32
C0 = 64
C1 = 128
C2 = 256
R2 = np.float32(0.2 ** 2)
BIGI = np.int32(100000)
EPS = np.float32(1e-5)
CNTF = np.float32(BN * NN * KS)

TS = 256
SB = 8
RT = 4096
NT = (NN * KS) // RT


def _gelu(x):
    return x * (np.float32(0.5) * (np.float32(1.0) + lax.erf(
        x * np.float32(1.0 / np.sqrt(2.0)))))


def _k0_body(xyz_ref, ptsT_ref, temb_ref, tw0_ref, tb0_ref, cw0_ref,
             cb0_ref, tw1_ref, tb1_ref, tw2_ref, tb2_ref,
             g_ref, rp_ref, t1_ref, t2_ref):
    xyz = xyz_ref[0]
    ptsT = ptsT_ref[0]
    te = temb_ref[0]
    ge = _gelu(te)
    dn = (((1,), (1,)), ((), ()))
    t0 = lax.dot_general(ge, tw0_ref[...], dn,
                         preferred_element_type=jnp.float32) + tb0_ref[...]
    cw0 = cw0_ref[...]
    cw0x = cw0[:, 0:3]
    cw0p = cw0[:, 3:131]
    gx = lax.dot_general(xyz, cw0x, dn, preferred_element_type=jnp.float32)
    gp = lax.dot_general(ptsT, cw0p, dn, preferred_element_type=jnp.float32)
    gt = lax.dot_general(t0, cw0, dn, preferred_element_type=jnp.float32)
    g_ref[0] = gx + gp + gt
    rp_ref[0] = gx - cb0_ref[...]
    t1 = lax.dot_general(ge, tw1_ref[...], dn,
                         preferred_element_type=jnp.float32) + tb1_ref[...]
    t2 = lax.dot_general(ge, tw2_ref[...], dn,
                         preferred_element_type=jnp.float32) + tb2_ref[...]
    t1_ref[0] = jnp.broadcast_to(t1, (8, C0))
    t2_ref[0] = jnp.broadcast_to(t2, (8, C1))


def _kmask_body(q_ref, xT_ref, rankp_ref, cnt_ref, first_ref):
    q = q_ref[0]
    xT = xT_ref[0]
    dx = q[:, 0:1] - xT[0:1, :]
    dy = q[:, 1:2] - xT[1:2, :]
    dz = q[:, 2:3] - xT[2:3, :]
    d2 = dx * dx + dy * dy + dz * dz
    mask = d2 <= R2
    mf = mask.astype(jnp.float32)
    r_in = lax.broadcasted_iota(jnp.int32, (128, 128), 0)
    c_in = lax.broadcasted_iota(jnp.int32, (128, 128), 1)
    tri_incl = (r_in <= c_in).astype(jnp.float32)
    mc = mf.reshape(TS * 16, 128)
    rank_in = jnp.dot(mc, tri_incl, preferred_element_type=jnp.float32)
    tot_c = mf.reshape(TS, 16, 128).sum(axis=2)
    r16 = lax.broadcasted_iota(jnp.int32, (16, 16), 0)
    c16 = lax.broadcasted_iota(jnp.int32, (16, 16), 1)
    tri_excl = (r16 < c16).astype(jnp.float32)
    pre = jnp.dot(tot_c, tri_excl, preferred_element_type=jnp.float32)
    rank = (rank_in.reshape(TS, 16, 128) + pre[:, :, None]).reshape(TS, NN)
    ranki = rank.astype(jnp.int32)
    sel = mask & (ranki <= KS)
    rankp_ref[0] = jnp.where(sel, ranki - 1, BIGI)
    total = tot_c.sum(axis=1).astype(jnp.int32)
    cnt = jnp.minimum(total, KS)
    iota_n = lax.broadcasted_iota(jnp.int32, (TS, NN), 1)
    first = jnp.min(jnp.where(mask, iota_n, BIGI), axis=1)
    cnt_ref[0] = jnp.broadcast_to(cnt[:, None], (TS, 8))
    first_ref[0] = jnp.broadcast_to(first[:, None], (TS, 8))


def _kgather_body(rankp_ref, g_ref, cnt_ref, first_ref, out_ref):
    rp = rankp_ref[0]
    g = g_ref[0]
    kio = lax.broadcasted_iota(jnp.int32, (SB, KS, NN), 1)
    oh = (rp[:, None, :] == kio).astype(jnp.float32).reshape(SB * KS, NN)
    gg = jnp.dot(oh, g, preferred_element_type=jnp.float32)
    fi = first_ref[0][:, 0:1]
    nio = lax.broadcasted_iota(jnp.int32, (SB, NN), 1)
    ohf = (nio == fi).astype(jnp.float32)
    gfirst = jnp.dot(ohf, g, preferred_element_type=jnp.float32)
    cnt = cnt_ref[0][:, 0:1]
    kio2 = lax.broadcasted_iota(jnp.int32, (SB, KS), 1)
    padm = (kio2 >= cnt).astype(jnp.float32)
    out = gg.reshape(SB, KS, C0) + padm[:, :, None] * gfirst[:, None, :]
    out_ref[0] = out.reshape(SB * KS, C0)


def _p1_body(gg_ref, rp_ref, st0_ref):
    g = gg_ref[0]
    r = rp_ref[0]
    y0 = g.reshape(RT // KS, KS, C0) - r[:, None, :]
    s = jnp.sum(y0, axis=(0, 1))
    q = jnp.sum(y0 * y0, axis=(0, 1))

    @pl.when((pl.program_id(0) == 0) & (pl.program_id(1) == 0))
    def _():
        st0_ref[...] = jnp.zeros_like(st0_ref)

    st0_ref[...] += jnp.concatenate([s[None, :], q[None, :]], axis=0)


def _bn_coefs(st_ref, g_w, b_w):
    s = st_ref[0, :]
    q = st_ref[1, :]
    mean = s / CNTF
    var = q / CNTF - mean * mean
    scale = g_w * lax.rsqrt(var + EPS)
    shift = b_w - mean * scale
    return scale, shift


def _p2_body(gg_ref, rp_ref, st0_ref, tv1_ref, cw1_ref, cb1_ref,
             g0_ref, b0_ref, y1_ref, st1_ref):
    scale0, shift0 = _bn_coefs(st0_ref, g0_ref[...], b0_ref[...])
    g = gg_ref[0]
    r = rp_ref[0]
    y0 = (g.reshape(RT // KS, KS, C0) - r[:, None, :]).reshape(RT, C0)
    x0 = _gelu(y0 * scale0 + shift0)
    a = x0 + tv1_ref[0][0:1, :]
    dn = (((1,), (1,)), ((), ()))
    y1 = lax.dot_general(a, cw1_ref[...], dn,
                         preferred_element_type=jnp.float32) + cb1_ref[...]
    y1_ref[0] = y1

    @pl.when((pl.program_id(0) == 0) & (pl.program_id(1) == 0))
    def _():
        st1_ref[...] = jnp.zeros_like(st1_ref)

    st1_ref[...] += jnp.concatenate(
        [jnp.sum(y1, axis=0)[None, :], jnp.sum(y1 * y1, axis=0)[None, :]], axis=0)


def _p3_body(y1_ref, st1_ref, tv2_ref, cw2_ref, cb2_ref,
             g1_ref, b1_ref, y2_ref, st2_ref):
    scale1, shift1 = _bn_coefs(st1_ref, g1_ref[...], b1_ref[...])
    y1 = y1_ref[0]
    x1 = _gelu(y1 * scale1 + shift1)
    a = x1 + tv2_ref[0][0:1, :]
    dn = (((1,), (1,)), ((), ()))
    y2 = lax.dot_general(a, cw2_ref[...], dn,
                         preferred_element_type=jnp.float32) + cb2_ref[...]
    y2_ref[0] = y2

    @pl.when((pl.program_id(0) == 0) & (pl.program_id(1) == 0))
    def _():
        st2_ref[...] = jnp.zeros_like(st2_ref)

    st2_ref[...] += jnp.concatenate(
        [jnp.sum(y2, axis=0)[None, :], jnp.sum(y2 * y2, axis=0)[None, :]], axis=0)


def _p4_body(y2_ref, st2_ref, g2_ref, b2_ref, out_ref):
    scale2, shift2 = _bn_coefs(st2_ref, g2_ref[...], b2_ref[...])
    y2 = y2_ref[0]
    x2 = _gelu(y2 * scale2 + shift2)
    m = jnp.max(x2.reshape(RT // KS, KS, C2), axis=1)
    out_ref[0] = m.T


def kernel(xyz, points, t_embed, tw0, tb0, cw0, cb0, g0, b0,
           tw1, tb1, cw1, cb1, g1, b1, tw2, tb2, cw2, cb2, g2, b2):
    f32 = jnp.float32
    ptsT = jnp.transpose(points, (0, 2, 1))
    xyzT = jnp.transpose(xyz, (0, 2, 1))
    temb3 = t_embed[:, None, :]

    g_tab, rp, tv1, tv2 = pl.pallas_call(
        _k0_body,
        grid=(BN,),
        in_specs=[
            pl.BlockSpec((1, NN, 3), lambda b: (b, 0, 0)),
            pl.BlockSpec((1, NN, 128), lambda b: (b, 0, 0)),
            pl.BlockSpec((1, 1, 256), lambda b: (b, 0, 0)),
            pl.BlockSpec((131, 256), lambda b: (0, 0)),
            pl.BlockSpec((131,), lambda b: (0,)),
            pl.BlockSpec((64, 131), lambda b: (0, 0)),
            pl.BlockSpec((64,), lambda b: (0,)),
            pl.BlockSpec((64, 256), lambda b: (0, 0)),
            pl.BlockSpec((64,), lambda b: (0,)),
            pl.BlockSpec((128, 256), lambda b: (0, 0)),
            pl.BlockSpec((128,), lambda b: (0,)),
        ],
        out_specs=[
            pl.BlockSpec((1, NN, C0), lambda b: (b, 0, 0)),
            pl.BlockSpec((1, NN, C0), lambda b: (b, 0, 0)),
            pl.BlockSpec((1, 8, C0), lambda b: (b, 0, 0)),
            pl.BlockSpec((1, 8, C1), lambda b: (b, 0, 0)),
        ],
        out_shape=[
            jax.ShapeDtypeStruct((BN, NN, C0), f32),
            jax.ShapeDtypeStruct((BN, NN, C0), f32),
            jax.ShapeDtypeStruct((BN, 8, C0), f32),
            jax.ShapeDtypeStruct((BN, 8, C1), f32),
        ],
    )(xyz, ptsT, temb3, tw0, tb0, cw0, cb0, tw1, tb1, tw2, tb2)

    rankp, cnt, first = pl.pallas_call(
        _kmask_body,
        grid=(BN, NN // TS),
        in_specs=[
            pl.BlockSpec((1, TS, 3), lambda b, j: (b, j, 0)),
            pl.BlockSpec((1, 3, NN), lambda b, j: (b, 0, 0)),
        ],
        out_specs=[
            pl.BlockSpec((1, TS, NN), lambda b, j: (b, j, 0)),
            pl.BlockSpec((1, TS, 8), lambda b, j: (b, j, 0)),
            pl.BlockSpec((1, TS, 8), lambda b, j: (b, j, 0)),
        ],
        out_shape=[
            jax.ShapeDtypeStruct((BN, NN, NN), jnp.int32),
            jax.ShapeDtypeStruct((BN, NN, 8), jnp.int32),
            jax.ShapeDtypeStruct((BN, NN, 8), jnp.int32),
        ],
    )(xyz, xyzT)

    ggath = pl.pallas_call(
        _kgather_body,
        grid=(BN, NN // SB),
        in_specs=[
            pl.BlockSpec((1, SB, NN), lambda b, j: (b, j, 0)),
            pl.BlockSpec((1, NN, C0), lambda b, j: (b, 0, 0)),
            pl.BlockSpec((1, SB, 8), lambda b, j: (b, j, 0)),
            pl.BlockSpec((1, SB, 8), lambda b, j: (b, j, 0)),
        ],
        out_specs=pl.BlockSpec((1, SB * KS, C0), lambda b, j: (b, j, 0)),
        out_shape=jax.ShapeDtypeStruct((BN, NN * KS, C0), f32),
    )(rankp, g_tab, cnt, first)

    st0 = pl.pallas_call(
        _p1_body,
        grid=(BN, NT),
        in_specs=[
            pl.BlockSpec((1, RT, C0), lambda b, t: (b, t, 0)),
            pl.BlockSpec((1, RT // KS, C0), lambda b, t: (b, t, 0)),
        ],
        out_specs=pl.BlockSpec((2, C0), lambda b, t: (0, 0)),
        out_shape=jax.ShapeDtypeStruct((2, C0), f32),
    )(ggath, rp)

    y1, st1 = pl.pallas_call(
        _p2_body,
        grid=(BN, NT),
        in_specs=[
            pl.BlockSpec((1, RT, C0), lambda b, t: (b, t, 0)),
            pl.BlockSpec((1, RT // KS, C0), lambda b, t: (b, t, 0)),
            pl.BlockSpec((2, C0), lambda b, t: (0, 0)),
            pl.BlockSpec((1, 8, C0), lambda b, t: (b, 0, 0)),
            pl.BlockSpec((C1, C0), lambda b, t: (0, 0)),
            pl.BlockSpec((C1,), lambda b, t: (0,)),
            pl.BlockSpec((C0,), lambda b, t: (0,)),
            pl.BlockSpec((C0,), lambda b, t: (0,)),
        ],
        out_specs=[
            pl.BlockSpec((1, RT, C1), lambda b, t: (b, t, 0)),
            pl.BlockSpec((2, C1), lambda b, t: (0, 0)),
        ],
        out_shape=[
            jax.ShapeDtypeStruct((BN, NN * KS, C1), f32),
            jax.ShapeDtypeStruct((2, C1), f32),
        ],
    )(ggath, rp, st0, tv1, cw1, cb1, g0, b0)

    y2, st2 = pl.pallas_call(
        _p3_body,
        grid=(BN, NT),
        in_specs=[
            pl.BlockSpec((1, RT, C1), lambda b, t: (b, t, 0)),
            pl.BlockSpec((2, C1), lambda b, t: (0, 0)),
            pl.BlockSpec((1, 8, C1), lambda b, t: (b, 0, 0)),
            pl.BlockSpec((C2, C1), lambda b, t: (0, 0)),
            pl.BlockSpec((C2,), lambda b, t: (0,)),
            pl.BlockSpec((C1,), lambda b, t: (0,)),
            pl.BlockSpec((C1,), lambda b, t: (0,)),
        ],
        out_specs=[
            pl.BlockSpec((1, RT, C2), lambda b, t: (b, t, 0)),
            pl.BlockSpec((2, C2), lambda b, t: (0, 0)),
        ],
        out_shape=[
            jax.ShapeDtypeStruct((BN, NN * KS, C2), f32),
            jax.ShapeDtypeStruct((2, C2), f32),
        ],
    )(y1, st1, tv2, cw2, cb2, g1, b1)

    out = pl.pallas_call(
        _p4_body,
        grid=(BN, NT),
        in_specs=[
            pl.BlockSpec((1, RT, C2), lambda b, t: (b, t, 0)),
            pl.BlockSpec((2, C2), lambda b, t: (0, 0)),
            pl.BlockSpec((C2,), lambda b, t: (0,)),
            pl.BlockSpec((C2,), lambda b, t: (0,)),
        ],
        out_specs=pl.BlockSpec((1, C2, RT // KS), lambda b, t: (b, 0, t)),
        out_shape=jax.ShapeDtypeStruct((BN, C2, NN), f32),
    )(y2, st2, g2, b2)

    return out


# speedup vs baseline: 10.0764x; 1.2378x over previous
import functools

import jax
import jax.numpy as jnp
import numpy as np
from jax import lax
from jax.experimental import pallas as pl
from jax.experimental.pallas import tpu as pltpu
from jax.experimental.pallas import tpu_sc as plsc

BN = 2
NN = 2048
KS = 32
C0 = 64
C1 = 128
C2 = 256
R2 = np.float32(0.2 ** 2)
BIGI = np.int32(100000)
EPS = np.float32(1e-5)
CNTF = np.float32(BN * NN * KS)

TS = 256
SB = 8
RT = 4096
NT = (NN * KS) // RT


def _gelu(x):
    return x * (np.float32(0.5) * (np.float32(1.0) + lax.erf(
        x * np.float32(1.0 / np.sqrt(2.0)))))


def _k0_body(xyz_ref, ptsT_ref, temb_ref, tw0_ref, tb0_ref, cw0_ref,
             cb0_ref, tw1_ref, tb1_ref, tw2_ref, tb2_ref,
             g_ref, rp_ref, t1_ref, t2_ref):
    xyz = xyz_ref[0]
    ptsT = ptsT_ref[0]
    te = temb_ref[0]
    ge = _gelu(te)
    dn = (((1,), (1,)), ((), ()))
    t0 = lax.dot_general(ge, tw0_ref[...], dn,
                         preferred_element_type=jnp.float32) + tb0_ref[...]
    cw0 = cw0_ref[...]
    cw0x = cw0[:, 0:3]
    cw0p = cw0[:, 3:131]
    gx = lax.dot_general(xyz, cw0x, dn, preferred_element_type=jnp.float32)
    gp = lax.dot_general(ptsT, cw0p, dn, preferred_element_type=jnp.float32)
    gt = lax.dot_general(t0, cw0, dn, preferred_element_type=jnp.float32)
    gval = gx + gp + gt
    g_ref[0] = jnp.concatenate(
        [gval, jnp.zeros((NN, C0), jnp.float32)], axis=1)
    rp_ref[0] = gx - cb0_ref[...]
    t1 = lax.dot_general(ge, tw1_ref[...], dn,
                         preferred_element_type=jnp.float32) + tb1_ref[...]
    t2 = lax.dot_general(ge, tw2_ref[...], dn,
                         preferred_element_type=jnp.float32) + tb2_ref[...]
    t1_ref[0] = jnp.broadcast_to(t1, (8, C0))
    t2_ref[0] = jnp.broadcast_to(t2, (8, C1))


def _kmask_body(q_ref, xT_ref, rankp_ref):
    q = q_ref[0]
    xT = xT_ref[0]
    dx = q[:, 0:1] - xT[0:1, :]
    dy = q[:, 1:2] - xT[1:2, :]
    dz = q[:, 2:3] - xT[2:3, :]
    d2 = dx * dx + dy * dy + dz * dz
    mask = d2 <= R2
    mf = mask.astype(jnp.float32)
    r_in = lax.broadcasted_iota(jnp.int32, (128, 128), 0)
    c_in = lax.broadcasted_iota(jnp.int32, (128, 128), 1)
    tri_incl = (r_in <= c_in).astype(jnp.float32)
    mc = mf.reshape(TS * 16, 128)
    rank_in = jnp.dot(mc, tri_incl, preferred_element_type=jnp.float32)
    tot_c = mf.reshape(TS, 16, 128).sum(axis=2)
    r16 = lax.broadcasted_iota(jnp.int32, (16, 16), 0)
    c16 = lax.broadcasted_iota(jnp.int32, (16, 16), 1)
    tri_excl = (r16 < c16).astype(jnp.float32)
    pre = jnp.dot(tot_c, tri_excl, preferred_element_type=jnp.float32)
    rank = (rank_in.reshape(TS, 16, 128) + pre[:, :, None]).reshape(TS, NN)
    ranki = rank.astype(jnp.int32)
    sel = mask & (ranki <= KS)
    rankp_ref[0] = jnp.where(sel, ranki - 1, BIGI)


NWORK = 32
RPW = (BN * NN) // NWORK
GR = 4
NG = RPW // GR
CPR = NN // 16


def _sc_gather_body(rankp_hbm, gtab_hbm, out_hbm, rbuf, gi_ref, idx_ref,
                    rows_ref, sem):
    cid = lax.axis_index("c")
    sid = lax.axis_index("s")
    wid = sid * 2 + cid
    row0 = wid * RPW
    boff = (wid // 16) * NN

    def group(g, carry):
        rowb = row0 + g * GR
        pltpu.sync_copy(rankp_hbm.at[pl.ds(rowb * NN, GR * NN)], rbuf)
        neg = jnp.full((16,), -1, jnp.int32)
        for j in range(GR * KS // 16):
            gi_ref[pl.ds(j * 16, 16)] = neg

        def chunk(i, c2):
            rv = rbuf[pl.ds(i * 16, 16)]
            m = rv < KS
            slot = rv + (i // CPR) * KS
            nv = lax.iota(jnp.int32, 16) + (i % CPR) * 16
            plsc.store_scatter(gi_ref, [slot], nv, mask=m)
            return c2

        lax.fori_loop(0, GR * CPR, chunk, 0, unroll=8)
        for r in range(GR):
            v0 = gi_ref[pl.ds(r * KS, 16)]
            v1 = gi_ref[pl.ds(r * KS + 16, 16)]
            m0 = v0 >= 0
            m1 = v1 >= 0
            c0 = jnp.where(m0, v0, BIGI)
            c1 = jnp.where(m1, v1, BIGI)
            mn = jnp.minimum(jnp.min(c0), jnp.min(c1))
            idx_ref[pl.ds(r * KS, 16)] = jnp.where(m0, v0, mn) + boff
            idx_ref[pl.ds(r * KS + 16, 16)] = jnp.where(m1, v1, mn) + boff
        pltpu.async_copy(gtab_hbm.at[idx_ref], rows_ref, sem).wait()
        pltpu.sync_copy(rows_ref, out_hbm.at[pl.ds(rowb * KS, GR * KS)])
        return carry

    lax.fori_loop(0, NG, group, 0)


def _p1_body(gg_ref, rp_ref, st0_ref):
    g = gg_ref[0][:, 0:C0]
    r = rp_ref[0]
    y0 = g.reshape(RT // KS, KS, C0) - r[:, None, :]
    s = jnp.sum(y0, axis=(0, 1))
    q = jnp.sum(y0 * y0, axis=(0, 1))

    @pl.when((pl.program_id(0) == 0) & (pl.program_id(1) == 0))
    def _():
        st0_ref[...] = jnp.zeros_like(st0_ref)

    st0_ref[...] += jnp.concatenate([s[None, :], q[None, :]], axis=0)


def _bn_coefs(st_ref, g_w, b_w):
    s = st_ref[0, :]
    q = st_ref[1, :]
    mean = s / CNTF
    var = q / CNTF - mean * mean
    scale = g_w * lax.rsqrt(var + EPS)
    shift = b_w - mean * scale
    return scale, shift


def _p2_body(gg_ref, rp_ref, st0_ref, tv1_ref, cw1_ref, cb1_ref,
             g0_ref, b0_ref, y1_ref, st1_ref):
    scale0, shift0 = _bn_coefs(st0_ref, g0_ref[...], b0_ref[...])
    g = gg_ref[0][:, 0:C0]
    r = rp_ref[0]
    y0 = (g.reshape(RT // KS, KS, C0) - r[:, None, :]).reshape(RT, C0)
    x0 = _gelu(y0 * scale0 + shift0)
    a = x0 + tv1_ref[0][0:1, :]
    dn = (((1,), (1,)), ((), ()))
    y1 = lax.dot_general(a, cw1_ref[...], dn,
                         preferred_element_type=jnp.float32) + cb1_ref[...]
    y1_ref[0] = y1

    @pl.when((pl.program_id(0) == 0) & (pl.program_id(1) == 0))
    def _():
        st1_ref[...] = jnp.zeros_like(st1_ref)

    st1_ref[...] += jnp.concatenate(
        [jnp.sum(y1, axis=0)[None, :], jnp.sum(y1 * y1, axis=0)[None, :]], axis=0)


def _p3_body(y1_ref, st1_ref, tv2_ref, cw2_ref, cb2_ref,
             g1_ref, b1_ref, y2_ref, st2_ref):
    scale1, shift1 = _bn_coefs(st1_ref, g1_ref[...], b1_ref[...])
    y1 = y1_ref[0]
    x1 = _gelu(y1 * scale1 + shift1)
    a = x1 + tv2_ref[0][0:1, :]
    dn = (((1,), (1,)), ((), ()))
    y2 = lax.dot_general(a, cw2_ref[...], dn,
                         preferred_element_type=jnp.float32) + cb2_ref[...]
    y2_ref[0] = y2

    @pl.when((pl.program_id(0) == 0) & (pl.program_id(1) == 0))
    def _():
        st2_ref[...] = jnp.zeros_like(st2_ref)

    st2_ref[...] += jnp.concatenate(
        [jnp.sum(y2, axis=0)[None, :], jnp.sum(y2 * y2, axis=0)[None, :]], axis=0)


def _p4_body(y2_ref, st2_ref, g2_ref, b2_ref, out_ref):
    scale2, shift2 = _bn_coefs(st2_ref, g2_ref[...], b2_ref[...])
    y2 = y2_ref[0]
    x2 = _gelu(y2 * scale2 + shift2)
    m = jnp.max(x2.reshape(RT // KS, KS, C2), axis=1)
    out_ref[0] = m.T


def kernel(xyz, points, t_embed, tw0, tb0, cw0, cb0, g0, b0,
           tw1, tb1, cw1, cb1, g1, b1, tw2, tb2, cw2, cb2, g2, b2):
    f32 = jnp.float32
    ptsT = jnp.transpose(points, (0, 2, 1))
    xyzT = jnp.transpose(xyz, (0, 2, 1))
    temb3 = t_embed[:, None, :]

    g_tab, rp, tv1, tv2 = pl.pallas_call(
        _k0_body,
        grid=(BN,),
        in_specs=[
            pl.BlockSpec((1, NN, 3), lambda b: (b, 0, 0)),
            pl.BlockSpec((1, NN, 128), lambda b: (b, 0, 0)),
            pl.BlockSpec((1, 1, 256), lambda b: (b, 0, 0)),
            pl.BlockSpec((131, 256), lambda b: (0, 0)),
            pl.BlockSpec((131,), lambda b: (0,)),
            pl.BlockSpec((64, 131), lambda b: (0, 0)),
            pl.BlockSpec((64,), lambda b: (0,)),
            pl.BlockSpec((64, 256), lambda b: (0, 0)),
            pl.BlockSpec((64,), lambda b: (0,)),
            pl.BlockSpec((128, 256), lambda b: (0, 0)),
            pl.BlockSpec((128,), lambda b: (0,)),
        ],
        out_specs=[
            pl.BlockSpec((1, NN, 2 * C0), lambda b: (b, 0, 0)),
            pl.BlockSpec((1, NN, C0), lambda b: (b, 0, 0)),
            pl.BlockSpec((1, 8, C0), lambda b: (b, 0, 0)),
            pl.BlockSpec((1, 8, C1), lambda b: (b, 0, 0)),
        ],
        out_shape=[
            jax.ShapeDtypeStruct((BN, NN, 2 * C0), f32),
            jax.ShapeDtypeStruct((BN, NN, C0), f32),
            jax.ShapeDtypeStruct((BN, 8, C0), f32),
            jax.ShapeDtypeStruct((BN, 8, C1), f32),
        ],
    )(xyz, ptsT, temb3, tw0, tb0, cw0, cb0, tw1, tb1, tw2, tb2)

    rankp = pl.pallas_call(
        _kmask_body,
        grid=(BN, NN // TS),
        in_specs=[
            pl.BlockSpec((1, TS, 3), lambda b, j: (b, j, 0)),
            pl.BlockSpec((1, 3, NN), lambda b, j: (b, 0, 0)),
        ],
        out_specs=pl.BlockSpec((1, TS, NN), lambda b, j: (b, j, 0)),
        out_shape=jax.ShapeDtypeStruct((BN, NN, NN), jnp.int32),
    )(xyz, xyzT)

    mesh = plsc.VectorSubcoreMesh(core_axis_name="c", subcore_axis_name="s")
    ggath_flat = pl.kernel(
        _sc_gather_body,
        out_type=jax.ShapeDtypeStruct((BN * NN * KS, 2 * C0), f32),
        mesh=mesh,
        compiler_params=pltpu.CompilerParams(needs_layout_passes=False),
        scratch_types=[
            pltpu.VMEM((GR * NN,), jnp.int32),
            pltpu.VMEM((GR * KS,), jnp.int32),
            pltpu.VMEM((GR * KS,), jnp.int32),
            pltpu.VMEM((GR * KS, 2 * C0), f32),
            pltpu.SemaphoreType.DMA,
        ],
    )(rankp.reshape(BN * NN * NN), g_tab.reshape(BN * NN, 2 * C0))
    ggath = ggath_flat.reshape(BN, NN * KS, 2 * C0)

    st0 = pl.pallas_call(
        _p1_body,
        grid=(BN, NT),
        in_specs=[
            pl.BlockSpec((1, RT, 2 * C0), lambda b, t: (b, t, 0)),
            pl.BlockSpec((1, RT // KS, C0), lambda b, t: (b, t, 0)),
        ],
        out_specs=pl.BlockSpec((2, C0), lambda b, t: (0, 0)),
        out_shape=jax.ShapeDtypeStruct((2, C0), f32),
    )(ggath, rp)

    y1, st1 = pl.pallas_call(
        _p2_body,
        grid=(BN, NT),
        in_specs=[
            pl.BlockSpec((1, RT, 2 * C0), lambda b, t: (b, t, 0)),
            pl.BlockSpec((1, RT // KS, C0), lambda b, t: (b, t, 0)),
            pl.BlockSpec((2, C0), lambda b, t: (0, 0)),
            pl.BlockSpec((1, 8, C0), lambda b, t: (b, 0, 0)),
            pl.BlockSpec((C1, C0), lambda b, t: (0, 0)),
            pl.BlockSpec((C1,), lambda b, t: (0,)),
            pl.BlockSpec((C0,), lambda b, t: (0,)),
            pl.BlockSpec((C0,), lambda b, t: (0,)),
        ],
        out_specs=[
            pl.BlockSpec((1, RT, C1), lambda b, t: (b, t, 0)),
            pl.BlockSpec((2, C1), lambda b, t: (0, 0)),
        ],
        out_shape=[
            jax.ShapeDtypeStruct((BN, NN * KS, C1), f32),
            jax.ShapeDtypeStruct((2, C1), f32),
        ],
    )(ggath, rp, st0, tv1, cw1, cb1, g0, b0)

    y2, st2 = pl.pallas_call(
        _p3_body,
        grid=(BN, NT),
        in_specs=[
            pl.BlockSpec((1, RT, C1), lambda b, t: (b, t, 0)),
            pl.BlockSpec((2, C1), lambda b, t: (0, 0)),
            pl.BlockSpec((1, 8, C1), lambda b, t: (b, 0, 0)),
            pl.BlockSpec((C2, C1), lambda b, t: (0, 0)),
            pl.BlockSpec((C2,), lambda b, t: (0,)),
            pl.BlockSpec((C1,), lambda b, t: (0,)),
            pl.BlockSpec((C1,), lambda b, t: (0,)),
        ],
        out_specs=[
            pl.BlockSpec((1, RT, C2), lambda b, t: (b, t, 0)),
            pl.BlockSpec((2, C2), lambda b, t: (0, 0)),
        ],
        out_shape=[
            jax.ShapeDtypeStruct((BN, NN * KS, C2), f32),
            jax.ShapeDtypeStruct((2, C2), f32),
        ],
    )(y1, st1, tv2, cw2, cb2, g1, b1)

    out = pl.pallas_call(
        _p4_body,
        grid=(BN, NT),
        in_specs=[
            pl.BlockSpec((1, RT, C2), lambda b, t: (b, t, 0)),
            pl.BlockSpec((2, C2), lambda b, t: (0, 0)),
            pl.BlockSpec((C2,), lambda b, t: (0,)),
            pl.BlockSpec((C2,), lambda b, t: (0,)),
        ],
        out_specs=pl.BlockSpec((1, C2, RT // KS), lambda b, t: (b, 0, t)),
        out_shape=jax.ShapeDtypeStruct((BN, C2, NN), f32),
    )(y2, st2, g2, b2)

    return out


# speedup vs baseline: 14.4335x; 1.4324x over previous
import functools

import jax
import jax.numpy as jnp
import numpy as np
from jax import lax
from jax.experimental import pallas as pl
from jax.experimental.pallas import tpu as pltpu
from jax.experimental.pallas import tpu_sc as plsc

BN = 2
NN = 2048
KS = 32
C0 = 64
C1 = 128
C2 = 256
R2 = np.float32(0.2 ** 2)
BIGI = np.int32(100000)
EPS = np.float32(1e-5)
CNTF = np.float32(BN * NN * KS)

TS = 256
SB = 8
RT = 4096
NT = (NN * KS) // RT


def _gelu(x):
    return x * (np.float32(0.5) * (np.float32(1.0) + lax.erf(
        x * np.float32(1.0 / np.sqrt(2.0)))))


def _k0_body(xyz_ref, ptsT_ref, temb_ref, tw0_ref, tb0_ref, cw0_ref,
             cb0_ref, tw1_ref, tb1_ref, tw2_ref, tb2_ref,
             g_ref, rp_ref, t1_ref, t2_ref):
    xyz = xyz_ref[0]
    ptsT = ptsT_ref[0]
    te = temb_ref[0]
    ge = _gelu(te)
    dn = (((1,), (1,)), ((), ()))
    t0 = lax.dot_general(ge, tw0_ref[...], dn,
                         preferred_element_type=jnp.float32) + tb0_ref[...]
    cw0 = cw0_ref[...]
    cw0x = cw0[:, 0:3]
    cw0p = cw0[:, 3:131]
    gx = lax.dot_general(xyz, cw0x, dn, preferred_element_type=jnp.float32)
    gp = lax.dot_general(ptsT, cw0p, dn, preferred_element_type=jnp.float32)
    gt = lax.dot_general(t0, cw0, dn, preferred_element_type=jnp.float32)
    gval = gx + gp + gt
    g_ref[0] = jnp.concatenate(
        [gval, jnp.zeros((NN, C0), jnp.float32)], axis=1)
    rp_ref[0] = gx - cb0_ref[...]
    t1 = lax.dot_general(ge, tw1_ref[...], dn,
                         preferred_element_type=jnp.float32) + tb1_ref[...]
    t2 = lax.dot_general(ge, tw2_ref[...], dn,
                         preferred_element_type=jnp.float32) + tb2_ref[...]
    t1_ref[0] = jnp.broadcast_to(t1, (8, C0))
    t2_ref[0] = jnp.broadcast_to(t2, (8, C1))


def _kmask_body(q_ref, xT_ref, rankp_ref):
    q = q_ref[0]
    xT = xT_ref[0]
    dx = q[:, 0:1] - xT[0:1, :]
    dy = q[:, 1:2] - xT[1:2, :]
    dz = q[:, 2:3] - xT[2:3, :]
    d2 = dx * dx + dy * dy + dz * dz
    mask = d2 <= R2
    mf = mask.astype(jnp.float32)
    r_in = lax.broadcasted_iota(jnp.int32, (128, 128), 0)
    c_in = lax.broadcasted_iota(jnp.int32, (128, 128), 1)
    tri_incl = (r_in <= c_in).astype(jnp.float32)
    mc = mf.reshape(TS * 16, 128)
    rank_in = jnp.dot(mc, tri_incl, preferred_element_type=jnp.float32)
    tot_c = mf.reshape(TS, 16, 128).sum(axis=2)
    r16 = lax.broadcasted_iota(jnp.int32, (16, 16), 0)
    c16 = lax.broadcasted_iota(jnp.int32, (16, 16), 1)
    tri_excl = (r16 < c16).astype(jnp.float32)
    pre = jnp.dot(tot_c, tri_excl, preferred_element_type=jnp.float32)
    rank = (rank_in.reshape(TS, 16, 128) + pre[:, :, None]).reshape(TS, NN)
    ranki = rank.astype(jnp.int32)
    sel = mask & (ranki <= KS)
    rankp_ref[...] = jnp.where(sel, ranki - 1, BIGI).reshape(TS * 16, 128)


NWORK = 32
RPW = (BN * NN) // NWORK
GR = 8
NG = RPW // GR
CPR = NN // 16
VR = NN // 128


def _sc_gather_body(rankp_hbm, gtab_hbm, out_hbm, rbuf, gi_ref, idx_ref,
                    rows_ref, semr0, semr1, semg0, semg1, semo0, semo1):
    cid = lax.axis_index("c")
    sid = lax.axis_index("s")
    wid = sid * 2 + cid
    row0 = wid * RPW
    boff = (wid // 16) * NN
    IL = GR * KS

    def issue_rank(g, h, sem):
        pltpu.async_copy(
            rankp_hbm.at[pl.ds((row0 + g * GR) * VR, GR * VR)],
            rbuf.at[pl.ds(h * GR * VR, GR * VR)], sem)

    def wait_rank(h, sem):
        pltpu.make_async_copy(
            rankp_hbm.at[pl.ds(0, GR * VR)],
            rbuf.at[pl.ds(h * GR * VR, GR * VR)], sem).wait()

    def issue_gather(h, sem):
        for q in range(IL // 128):
            pltpu.async_copy(
                gtab_hbm.at[idx_ref.at[pl.ds(h * IL + q * 128, 128)]],
                rows_ref.at[pl.ds(h * IL + q * 128, 128)], sem)

    def wait_gather(h, sem):
        for q in range(IL // 128):
            pltpu.make_async_copy(
                gtab_hbm.at[idx_ref.at[pl.ds(h * IL + q * 128, 128)]],
                rows_ref.at[pl.ds(h * IL + q * 128, 128)], sem).wait()

    def issue_out(g, h, sem):
        pltpu.async_copy(rows_ref.at[pl.ds(h * IL, IL)],
                         out_hbm.at[pl.ds((row0 + g * GR) * KS, IL)], sem)

    def wait_out(h, sem):
        pltpu.make_async_copy(rows_ref.at[pl.ds(h * IL, IL)],
                              out_hbm.at[pl.ds(0, IL)], sem).wait()

    def compact(h):
        base_r = h * GR * VR
        neg = jnp.full((16,), -1, jnp.int32)
        for j in range(IL // 16):
            gi_ref[pl.ds(j * 16, 16)] = neg

        @plsc.parallel_loop(0, GR * CPR, 1, unroll=8)
        def _(i):
            r = i // CPR
            c = i % CPR
            rv = rbuf[base_r + r * VR + c // 8, pl.ds((c % 8) * 16, 16)]
            m = rv < KS
            slot = rv + r * KS
            nv = lax.iota(jnp.int32, 16) + c * 16
            plsc.store_scatter(gi_ref, [slot], nv, mask=m)

        for r in range(GR):
            v0 = gi_ref[pl.ds(r * KS, 16)]
            v1 = gi_ref[pl.ds(r * KS + 16, 16)]
            m0 = v0 >= 0
            m1 = v1 >= 0
            c0 = jnp.where(m0, v0, BIGI)
            c1 = jnp.where(m1, v1, BIGI)
            mn = jnp.minimum(jnp.min(c0), jnp.min(c1))
            idx_ref[pl.ds(h * IL + r * KS, 16)] = jnp.where(m0, v0, mn) + boff
            idx_ref[pl.ds(h * IL + r * KS + 16, 16)] = \
                jnp.where(m1, v1, mn) + boff

    issue_rank(0, 0, semr0)
    issue_rank(1, 1, semr1)

    def body(it, carry):
        for h, semr, semg, semo in ((0, semr0, semg0, semo0),
                                    (1, semr1, semg1, semo1)):
            g = it * 2 + h
            wait_rank(h, semr)

            @pl.when(it > 0)
            def _():
                wait_gather(h, semg)
                issue_out(g - 2, h, semo)

            compact(h)

            @pl.when(it > 0)
            def _():
                wait_out(h, semo)

            issue_gather(h, semg)

            @pl.when(g + 2 < NG)
            def _():
                issue_rank(g + 2, h, semr)
        return carry

    lax.fori_loop(0, NG // 2, body, 0)
    for h, semg, semo in ((0, semg0, semo0), (1, semg1, semo1)):
        wait_gather(h, semg)
        issue_out(NG - 2 + h, h, semo)
        wait_out(h, semo)


def _sc_call(rankp, gtab):
    mesh = plsc.VectorSubcoreMesh(core_axis_name="c", subcore_axis_name="s")
    return pl.kernel(
        _sc_gather_body,
        out_type=jax.ShapeDtypeStruct((BN * NN * KS, 2 * C0), jnp.float32),
        mesh=mesh,
        compiler_params=pltpu.CompilerParams(needs_layout_passes=False),
        scratch_types=[
            pltpu.VMEM((2 * GR * VR, 128), jnp.int32),
            pltpu.VMEM((GR * KS,), jnp.int32),
            pltpu.VMEM((2 * GR * KS,), jnp.int32),
            pltpu.VMEM((2 * GR * KS, 2 * C0), jnp.float32),
            pltpu.SemaphoreType.DMA,
            pltpu.SemaphoreType.DMA,
            pltpu.SemaphoreType.DMA,
            pltpu.SemaphoreType.DMA,
            pltpu.SemaphoreType.DMA,
            pltpu.SemaphoreType.DMA,
        ],
    )(rankp, gtab)


def _p1_body(gg_ref, rp_ref, st0_ref):
    g = gg_ref[0][:, 0:C0]
    r = rp_ref[0]
    y0 = g.reshape(RT // KS, KS, C0) - r[:, None, :]
    s = jnp.sum(y0, axis=(0, 1))
    q = jnp.sum(y0 * y0, axis=(0, 1))

    @pl.when((pl.program_id(0) == 0) & (pl.program_id(1) == 0))
    def _():
        st0_ref[...] = jnp.zeros_like(st0_ref)

    st0_ref[...] += jnp.concatenate([s[None, :], q[None, :]], axis=0)


def _bn_coefs(st_ref, g_w, b_w):
    s = st_ref[0, :]
    q = st_ref[1, :]
    mean = s / CNTF
    var = q / CNTF - mean * mean
    scale = g_w * lax.rsqrt(var + EPS)
    shift = b_w - mean * scale
    return scale, shift


def _p2_body(gg_ref, rp_ref, st0_ref, tv1_ref, cw1_ref, cb1_ref,
             g0_ref, b0_ref, y1_ref, st1_ref):
    scale0, shift0 = _bn_coefs(st0_ref, g0_ref[...], b0_ref[...])
    g = gg_ref[0][:, 0:C0]
    r = rp_ref[0]
    y0 = (g.reshape(RT // KS, KS, C0) - r[:, None, :]).reshape(RT, C0)
    x0 = _gelu(y0 * scale0 + shift0)
    a = x0 + tv1_ref[0][0:1, :]
    dn = (((1,), (1,)), ((), ()))
    y1 = lax.dot_general(a, cw1_ref[...], dn,
                         preferred_element_type=jnp.float32) + cb1_ref[...]
    y1_ref[0] = y1

    @pl.when((pl.program_id(0) == 0) & (pl.program_id(1) == 0))
    def _():
        st1_ref[...] = jnp.zeros_like(st1_ref)

    st1_ref[...] += jnp.concatenate(
        [jnp.sum(y1, axis=0)[None, :], jnp.sum(y1 * y1, axis=0)[None, :]], axis=0)


def _p3_body(y1_ref, st1_ref, tv2_ref, cw2_ref, cb2_ref,
             g1_ref, b1_ref, y2_ref, st2_ref):
    scale1, shift1 = _bn_coefs(st1_ref, g1_ref[...], b1_ref[...])
    y1 = y1_ref[0]
    x1 = _gelu(y1 * scale1 + shift1)
    a = x1 + tv2_ref[0][0:1, :]
    dn = (((1,), (1,)), ((), ()))
    y2 = lax.dot_general(a, cw2_ref[...], dn,
                         preferred_element_type=jnp.float32) + cb2_ref[...]
    y2_ref[0] = y2

    @pl.when((pl.program_id(0) == 0) & (pl.program_id(1) == 0))
    def _():
        st2_ref[...] = jnp.zeros_like(st2_ref)

    st2_ref[...] += jnp.concatenate(
        [jnp.sum(y2, axis=0)[None, :], jnp.sum(y2 * y2, axis=0)[None, :]], axis=0)


def _p4_body(y2_ref, st2_ref, g2_ref, b2_ref, out_ref):
    scale2, shift2 = _bn_coefs(st2_ref, g2_ref[...], b2_ref[...])
    y2 = y2_ref[0]
    x2 = _gelu(y2 * scale2 + shift2)
    m = jnp.max(x2.reshape(RT // KS, KS, C2), axis=1)
    out_ref[0] = m.T


def kernel(xyz, points, t_embed, tw0, tb0, cw0, cb0, g0, b0,
           tw1, tb1, cw1, cb1, g1, b1, tw2, tb2, cw2, cb2, g2, b2):
    f32 = jnp.float32
    ptsT = jnp.transpose(points, (0, 2, 1))
    xyzT = jnp.transpose(xyz, (0, 2, 1))
    temb3 = t_embed[:, None, :]

    g_tab, rp, tv1, tv2 = pl.pallas_call(
        _k0_body,
        grid=(BN,),
        in_specs=[
            pl.BlockSpec((1, NN, 3), lambda b: (b, 0, 0)),
            pl.BlockSpec((1, NN, 128), lambda b: (b, 0, 0)),
            pl.BlockSpec((1, 1, 256), lambda b: (b, 0, 0)),
            pl.BlockSpec((131, 256), lambda b: (0, 0)),
            pl.BlockSpec((131,), lambda b: (0,)),
            pl.BlockSpec((64, 131), lambda b: (0, 0)),
            pl.BlockSpec((64,), lambda b: (0,)),
            pl.BlockSpec((64, 256), lambda b: (0, 0)),
            pl.BlockSpec((64,), lambda b: (0,)),
            pl.BlockSpec((128, 256), lambda b: (0, 0)),
            pl.BlockSpec((128,), lambda b: (0,)),
        ],
        out_specs=[
            pl.BlockSpec((1, NN, 2 * C0), lambda b: (b, 0, 0)),
            pl.BlockSpec((1, NN, C0), lambda b: (b, 0, 0)),
            pl.BlockSpec((1, 8, C0), lambda b: (b, 0, 0)),
            pl.BlockSpec((1, 8, C1), lambda b: (b, 0, 0)),
        ],
        out_shape=[
            jax.ShapeDtypeStruct((BN, NN, 2 * C0), f32),
            jax.ShapeDtypeStruct((BN, NN, C0), f32),
            jax.ShapeDtypeStruct((BN, 8, C0), f32),
            jax.ShapeDtypeStruct((BN, 8, C1), f32),
        ],
    )(xyz, ptsT, temb3, tw0, tb0, cw0, cb0, tw1, tb1, tw2, tb2)

    rankp = pl.pallas_call(
        _kmask_body,
        grid=(BN, NN // TS),
        in_specs=[
            pl.BlockSpec((1, TS, 3), lambda b, j: (b, j, 0)),
            pl.BlockSpec((1, 3, NN), lambda b, j: (b, 0, 0)),
        ],
        out_specs=pl.BlockSpec((TS * 16, 128), lambda b, j: (b * 8 + j, 0)),
        out_shape=jax.ShapeDtypeStruct((BN * NN * 16, 128), jnp.int32),
    )(xyz, xyzT)

    ggath_flat = _sc_call(rankp, g_tab.reshape(BN * NN, 2 * C0))
    ggath = ggath_flat.reshape(BN, NN * KS, 2 * C0)

    st0 = pl.pallas_call(
        _p1_body,
        grid=(BN, NT),
        in_specs=[
            pl.BlockSpec((1, RT, 2 * C0), lambda b, t: (b, t, 0)),
            pl.BlockSpec((1, RT // KS, C0), lambda b, t: (b, t, 0)),
        ],
        out_specs=pl.BlockSpec((2, C0), lambda b, t: (0, 0)),
        out_shape=jax.ShapeDtypeStruct((2, C0), f32),
    )(ggath, rp)

    y1, st1 = pl.pallas_call(
        _p2_body,
        grid=(BN, NT),
        in_specs=[
            pl.BlockSpec((1, RT, 2 * C0), lambda b, t: (b, t, 0)),
            pl.BlockSpec((1, RT // KS, C0), lambda b, t: (b, t, 0)),
            pl.BlockSpec((2, C0), lambda b, t: (0, 0)),
            pl.BlockSpec((1, 8, C0), lambda b, t: (b, 0, 0)),
            pl.BlockSpec((C1, C0), lambda b, t: (0, 0)),
            pl.BlockSpec((C1,), lambda b, t: (0,)),
            pl.BlockSpec((C0,), lambda b, t: (0,)),
            pl.BlockSpec((C0,), lambda b, t: (0,)),
        ],
        out_specs=[
            pl.BlockSpec((1, RT, C1), lambda b, t: (b, t, 0)),
            pl.BlockSpec((2, C1), lambda b, t: (0, 0)),
        ],
        out_shape=[
            jax.ShapeDtypeStruct((BN, NN * KS, C1), f32),
            jax.ShapeDtypeStruct((2, C1), f32),
        ],
    )(ggath, rp, st0, tv1, cw1, cb1, g0, b0)

    y2, st2 = pl.pallas_call(
        _p3_body,
        grid=(BN, NT),
        in_specs=[
            pl.BlockSpec((1, RT, C1), lambda b, t: (b, t, 0)),
            pl.BlockSpec((2, C1), lambda b, t: (0, 0)),
            pl.BlockSpec((1, 8, C1), lambda b, t: (b, 0, 0)),
            pl.BlockSpec((C2, C1), lambda b, t: (0, 0)),
            pl.BlockSpec((C2,), lambda b, t: (0,)),
            pl.BlockSpec((C1,), lambda b, t: (0,)),
            pl.BlockSpec((C1,), lambda b, t: (0,)),
        ],
        out_specs=[
            pl.BlockSpec((1, RT, C2), lambda b, t: (b, t, 0)),
            pl.BlockSpec((2, C2), lambda b, t: (0, 0)),
        ],
        out_shape=[
            jax.ShapeDtypeStruct((BN, NN * KS, C2), f32),
            jax.ShapeDtypeStruct((2, C2), f32),
        ],
    )(y1, st1, tv2, cw2, cb2, g1, b1)

    out = pl.pallas_call(
        _p4_body,
        grid=(BN, NT),
        in_specs=[
            pl.BlockSpec((1, RT, C2), lambda b, t: (b, t, 0)),
            pl.BlockSpec((2, C2), lambda b, t: (0, 0)),
            pl.BlockSpec((C2,), lambda b, t: (0,)),
            pl.BlockSpec((C2,), lambda b, t: (0,)),
        ],
        out_specs=pl.BlockSpec((1, C2, RT // KS), lambda b, t: (b, 0, t)),
        out_shape=jax.ShapeDtypeStruct((BN, C2, NN), f32),
    )(y2, st2, g2, b2)

    return out


# speedup vs baseline: 15.6213x; 1.0823x over previous
import functools

import jax
import jax.numpy as jnp
import numpy as np
from jax import lax
from jax.experimental import pallas as pl
from jax.experimental.pallas import tpu as pltpu
from jax.experimental.pallas import tpu_sc as plsc

BN = 2
NN = 2048
KS = 32
C0 = 64
C1 = 128
C2 = 256
R2 = np.float32(0.2 ** 2)
BIGI = np.int32(100000)
EPS = np.float32(1e-5)
CNTF = np.float32(BN * NN * KS)

TS = 256
SB = 8
RT = 4096
NT = (NN * KS) // RT


def _gelu(x):
    return x * (np.float32(0.5) * (np.float32(1.0) + lax.erf(
        x * np.float32(1.0 / np.sqrt(2.0)))))


def _k0_body(xyz_ref, ptsT_ref, temb_ref, tw0_ref, tb0_ref, cw0_ref,
             cb0_ref, tw1_ref, tb1_ref, tw2_ref, tb2_ref,
             g_ref, rp_ref, t1_ref, t2_ref):
    xyz = xyz_ref[0]
    ptsT = ptsT_ref[0]
    te = temb_ref[0]
    ge = _gelu(te)
    dn = (((1,), (1,)), ((), ()))
    t0 = lax.dot_general(ge, tw0_ref[...], dn,
                         preferred_element_type=jnp.float32) + tb0_ref[...]
    cw0 = cw0_ref[...]
    cw0x = cw0[:, 0:3]
    cw0p = cw0[:, 3:131]
    gx = lax.dot_general(xyz, cw0x, dn, preferred_element_type=jnp.float32)
    gp = lax.dot_general(ptsT, cw0p, dn, preferred_element_type=jnp.float32)
    gt = lax.dot_general(t0, cw0, dn, preferred_element_type=jnp.float32)
    gval = gx + gp + gt
    g_ref[0] = jnp.concatenate(
        [gval, jnp.zeros((NN, C0), jnp.float32)], axis=1)
    rp_ref[0] = gx - cb0_ref[...]
    t1 = lax.dot_general(ge, tw1_ref[...], dn,
                         preferred_element_type=jnp.float32) + tb1_ref[...]
    t2 = lax.dot_general(ge, tw2_ref[...], dn,
                         preferred_element_type=jnp.float32) + tb2_ref[...]
    t1_ref[0] = jnp.broadcast_to(t1, (8, C0))
    t2_ref[0] = jnp.broadcast_to(t2, (8, C1))


def _kmask_body(q_ref, xT_ref, rankp_ref):
    q = q_ref[0]
    xT = xT_ref[0]
    dx = q[:, 0:1] - xT[0:1, :]
    dy = q[:, 1:2] - xT[1:2, :]
    dz = q[:, 2:3] - xT[2:3, :]
    d2 = dx * dx + dy * dy + dz * dz
    mask = d2 <= R2
    mf = mask.astype(jnp.float32)
    r_in = lax.broadcasted_iota(jnp.int32, (128, 128), 0)
    c_in = lax.broadcasted_iota(jnp.int32, (128, 128), 1)
    tri_incl = (r_in <= c_in).astype(jnp.float32)
    mc = mf.reshape(TS * 16, 128)
    rank_in = jnp.dot(mc, tri_incl, preferred_element_type=jnp.float32)
    tot_c = mf.reshape(TS, 16, 128).sum(axis=2)
    r16 = lax.broadcasted_iota(jnp.int32, (16, 16), 0)
    c16 = lax.broadcasted_iota(jnp.int32, (16, 16), 1)
    tri_excl = (r16 < c16).astype(jnp.float32)
    pre = jnp.dot(tot_c, tri_excl, preferred_element_type=jnp.float32)
    rank = (rank_in.reshape(TS, 16, 128) + pre[:, :, None]).reshape(TS, NN)
    ranki = rank.astype(jnp.int32)
    sel = mask & (ranki <= KS)
    rankp_ref[...] = jnp.where(sel, ranki - 1, BIGI).reshape(TS * 16, 128)


NWORK = 32
RPW = (BN * NN) // NWORK
GR = 8
NG = RPW // GR
CPR = NN // 16
VR = NN // 128


def _sc_gather_body(rankp_hbm, gtab_hbm, out_hbm, rbuf, gi_ref, idx_ref,
                    rows_ref, semr0, semr1, semg0, semg1, semo0, semo1):
    cid = lax.axis_index("c")
    sid = lax.axis_index("s")
    wid = sid * 2 + cid
    row0 = wid * RPW
    boff = (wid // 16) * NN
    IL = GR * KS

    def issue_rank(g, h, sem):
        pltpu.async_copy(
            rankp_hbm.at[pl.ds((row0 + g * GR) * VR, GR * VR)],
            rbuf.at[pl.ds(h * GR * VR, GR * VR)], sem)

    def wait_rank(h, sem):
        pltpu.make_async_copy(
            rankp_hbm.at[pl.ds(0, GR * VR)],
            rbuf.at[pl.ds(h * GR * VR, GR * VR)], sem).wait()

    def issue_gather(h, sem):
        for q in range(IL // 128):
            pltpu.async_copy(
                gtab_hbm.at[idx_ref.at[pl.ds(h * IL + q * 128, 128)]],
                rows_ref.at[pl.ds(h * IL + q * 128, 128)], sem)

    def wait_gather(h, sem):
        for q in range(IL // 128):
            pltpu.make_async_copy(
                gtab_hbm.at[idx_ref.at[pl.ds(h * IL + q * 128, 128)]],
                rows_ref.at[pl.ds(h * IL + q * 128, 128)], sem).wait()

    def issue_out(g, h, sem):
        pltpu.async_copy(rows_ref.at[pl.ds(h * IL, IL)],
                         out_hbm.at[pl.ds((row0 + g * GR) * KS, IL)], sem)

    def wait_out(h, sem):
        pltpu.make_async_copy(rows_ref.at[pl.ds(h * IL, IL)],
                              out_hbm.at[pl.ds(0, IL)], sem).wait()

    def compact(h):
        base_r = h * GR * VR
        neg = jnp.full((16,), -1, jnp.int32)
        for j in range(IL // 16):
            gi_ref[pl.ds(j * 16, 16)] = neg

        @plsc.parallel_loop(0, GR * CPR, 1, unroll=8)
        def _(i):
            r = i // CPR
            c = i % CPR
            rv = rbuf[base_r + r * VR + c // 8, pl.ds((c % 8) * 16, 16)]
            m = rv < KS
            slot = rv + r * KS
            nv = lax.iota(jnp.int32, 16) + c * 16
            plsc.store_scatter(gi_ref, [slot], nv, mask=m)

        for r in range(GR):
            v0 = gi_ref[pl.ds(r * KS, 16)]
            v1 = gi_ref[pl.ds(r * KS + 16, 16)]
            m0 = v0 >= 0
            m1 = v1 >= 0
            c0 = jnp.where(m0, v0, BIGI)
            c1 = jnp.where(m1, v1, BIGI)
            mn = jnp.minimum(jnp.min(c0), jnp.min(c1))
            idx_ref[pl.ds(h * IL + r * KS, 16)] = jnp.where(m0, v0, mn) + boff
            idx_ref[pl.ds(h * IL + r * KS + 16, 16)] = \
                jnp.where(m1, v1, mn) + boff

    issue_rank(0, 0, semr0)
    issue_rank(1, 1, semr1)

    def body(it, carry):
        for h, semr, semg, semo in ((0, semr0, semg0, semo0),
                                    (1, semr1, semg1, semo1)):
            g = it * 2 + h
            wait_rank(h, semr)

            @pl.when(it > 0)
            def _():
                wait_gather(h, semg)
                issue_out(g - 2, h, semo)

            compact(h)

            @pl.when(it > 0)
            def _():
                wait_out(h, semo)

            issue_gather(h, semg)

            @pl.when(g + 2 < NG)
            def _():
                issue_rank(g + 2, h, semr)
        return carry

    lax.fori_loop(0, NG // 2, body, 0)
    for h, semg, semo in ((0, semg0, semo0), (1, semg1, semo1)):
        wait_gather(h, semg)
        issue_out(NG - 2 + h, h, semo)
        wait_out(h, semo)


def _sc_call(rankp, gtab):
    mesh = plsc.VectorSubcoreMesh(core_axis_name="c", subcore_axis_name="s")
    return pl.kernel(
        _sc_gather_body,
        out_type=jax.ShapeDtypeStruct((BN * NN * KS, 2 * C0), jnp.float32),
        mesh=mesh,
        compiler_params=pltpu.CompilerParams(needs_layout_passes=False),
        scratch_types=[
            pltpu.VMEM((2 * GR * VR, 128), jnp.int32),
            pltpu.VMEM((GR * KS,), jnp.int32),
            pltpu.VMEM((2 * GR * KS,), jnp.int32),
            pltpu.VMEM((2 * GR * KS, 2 * C0), jnp.float32),
            pltpu.SemaphoreType.DMA,
            pltpu.SemaphoreType.DMA,
            pltpu.SemaphoreType.DMA,
            pltpu.SemaphoreType.DMA,
            pltpu.SemaphoreType.DMA,
            pltpu.SemaphoreType.DMA,
        ],
    )(rankp, gtab)


def _p1_body(gg_ref, rp_ref, st0_ref):
    g = gg_ref[0][:, 0:C0]
    r = rp_ref[0]
    y0 = g.reshape(RT // KS, KS, C0) - r[:, None, :]
    s = jnp.sum(y0, axis=(0, 1))
    q = jnp.sum(y0 * y0, axis=(0, 1))

    @pl.when((pl.program_id(0) == 0) & (pl.program_id(1) == 0))
    def _():
        st0_ref[...] = jnp.zeros_like(st0_ref)

    st0_ref[...] += jnp.concatenate([s[None, :], q[None, :]], axis=0)


def _bn_coefs(st_ref, g_w, b_w):
    s = st_ref[0, :]
    q = st_ref[1, :]
    mean = s / CNTF
    var = q / CNTF - mean * mean
    scale = g_w * lax.rsqrt(var + EPS)
    shift = b_w - mean * scale
    return scale, shift


def _p2_body(gg_ref, rp_ref, st0_ref, tv1_ref, cw1_ref, cb1_ref,
             g0_ref, b0_ref, y1_ref, st1_ref):
    scale0, shift0 = _bn_coefs(st0_ref, g0_ref[...], b0_ref[...])
    g = gg_ref[0][:, 0:C0]
    r = rp_ref[0]
    y0 = (g.reshape(RT // KS, KS, C0) - r[:, None, :]).reshape(RT, C0)
    x0 = _gelu(y0 * scale0 + shift0)
    a = x0 + tv1_ref[0][0:1, :]
    dn = (((1,), (1,)), ((), ()))
    y1 = lax.dot_general(a, cw1_ref[...], dn,
                         preferred_element_type=jnp.float32) + cb1_ref[...]
    y1b = y1.astype(jnp.bfloat16)
    y1_ref[0] = y1b
    y1s = y1b.astype(jnp.float32)

    @pl.when((pl.program_id(0) == 0) & (pl.program_id(1) == 0))
    def _():
        st1_ref[...] = jnp.zeros_like(st1_ref)

    st1_ref[...] += jnp.concatenate(
        [jnp.sum(y1s, axis=0)[None, :], jnp.sum(y1s * y1s, axis=0)[None, :]],
        axis=0)


def _p3_body(y1_ref, st1_ref, tv2_ref, cw2_ref, cb2_ref,
             g1_ref, b1_ref, y2_ref, st2_ref):
    scale1, shift1 = _bn_coefs(st1_ref, g1_ref[...], b1_ref[...])
    y1 = y1_ref[0].astype(jnp.float32)
    x1 = _gelu(y1 * scale1 + shift1)
    a = x1 + tv2_ref[0][0:1, :]
    dn = (((1,), (1,)), ((), ()))
    y2 = lax.dot_general(a, cw2_ref[...], dn,
                         preferred_element_type=jnp.float32) + cb2_ref[...]
    y2b = y2.astype(jnp.bfloat16)
    y2_ref[0] = y2b
    y2s = y2b.astype(jnp.float32)

    @pl.when((pl.program_id(0) == 0) & (pl.program_id(1) == 0))
    def _():
        st2_ref[...] = jnp.zeros_like(st2_ref)

    st2_ref[...] += jnp.concatenate(
        [jnp.sum(y2s, axis=0)[None, :], jnp.sum(y2s * y2s, axis=0)[None, :]],
        axis=0)


def _p4_body(y2_ref, st2_ref, g2_ref, b2_ref, out_ref):
    scale2, shift2 = _bn_coefs(st2_ref, g2_ref[...], b2_ref[...])
    y2 = y2_ref[0].astype(jnp.float32)
    z = (y2 * scale2 + shift2).reshape(RT // KS, KS, C2)
    zmx = jnp.max(z, axis=1)
    zmn = jnp.min(z, axis=1)
    m = jnp.maximum(_gelu(zmx), _gelu(zmn))
    out_ref[0] = m.T


def kernel(xyz, points, t_embed, tw0, tb0, cw0, cb0, g0, b0,
           tw1, tb1, cw1, cb1, g1, b1, tw2, tb2, cw2, cb2, g2, b2):
    f32 = jnp.float32
    ptsT = jnp.transpose(points, (0, 2, 1))
    xyzT = jnp.transpose(xyz, (0, 2, 1))
    temb3 = t_embed[:, None, :]

    g_tab, rp, tv1, tv2 = pl.pallas_call(
        _k0_body,
        grid=(BN,),
        in_specs=[
            pl.BlockSpec((1, NN, 3), lambda b: (b, 0, 0)),
            pl.BlockSpec((1, NN, 128), lambda b: (b, 0, 0)),
            pl.BlockSpec((1, 1, 256), lambda b: (b, 0, 0)),
            pl.BlockSpec((131, 256), lambda b: (0, 0)),
            pl.BlockSpec((131,), lambda b: (0,)),
            pl.BlockSpec((64, 131), lambda b: (0, 0)),
            pl.BlockSpec((64,), lambda b: (0,)),
            pl.BlockSpec((64, 256), lambda b: (0, 0)),
            pl.BlockSpec((64,), lambda b: (0,)),
            pl.BlockSpec((128, 256), lambda b: (0, 0)),
            pl.BlockSpec((128,), lambda b: (0,)),
        ],
        out_specs=[
            pl.BlockSpec((1, NN, 2 * C0), lambda b: (b, 0, 0)),
            pl.BlockSpec((1, NN, C0), lambda b: (b, 0, 0)),
            pl.BlockSpec((1, 8, C0), lambda b: (b, 0, 0)),
            pl.BlockSpec((1, 8, C1), lambda b: (b, 0, 0)),
        ],
        out_shape=[
            jax.ShapeDtypeStruct((BN, NN, 2 * C0), f32),
            jax.ShapeDtypeStruct((BN, NN, C0), f32),
            jax.ShapeDtypeStruct((BN, 8, C0), f32),
            jax.ShapeDtypeStruct((BN, 8, C1), f32),
        ],
    )(xyz, ptsT, temb3, tw0, tb0, cw0, cb0, tw1, tb1, tw2, tb2)

    rankp = pl.pallas_call(
        _kmask_body,
        grid=(BN, NN // TS),
        in_specs=[
            pl.BlockSpec((1, TS, 3), lambda b, j: (b, j, 0)),
            pl.BlockSpec((1, 3, NN), lambda b, j: (b, 0, 0)),
        ],
        out_specs=pl.BlockSpec((TS * 16, 128), lambda b, j: (b * 8 + j, 0)),
        out_shape=jax.ShapeDtypeStruct((BN * NN * 16, 128), jnp.int32),
    )(xyz, xyzT)

    ggath_flat = _sc_call(rankp, g_tab.reshape(BN * NN, 2 * C0))
    ggath = ggath_flat.reshape(BN, NN * KS, 2 * C0)

    st0 = pl.pallas_call(
        _p1_body,
        grid=(BN, NT),
        in_specs=[
            pl.BlockSpec((1, RT, 2 * C0), lambda b, t: (b, t, 0)),
            pl.BlockSpec((1, RT // KS, C0), lambda b, t: (b, t, 0)),
        ],
        out_specs=pl.BlockSpec((2, C0), lambda b, t: (0, 0)),
        out_shape=jax.ShapeDtypeStruct((2, C0), f32),
    )(ggath, rp)

    y1, st1 = pl.pallas_call(
        _p2_body,
        grid=(BN, NT),
        in_specs=[
            pl.BlockSpec((1, RT, 2 * C0), lambda b, t: (b, t, 0)),
            pl.BlockSpec((1, RT // KS, C0), lambda b, t: (b, t, 0)),
            pl.BlockSpec((2, C0), lambda b, t: (0, 0)),
            pl.BlockSpec((1, 8, C0), lambda b, t: (b, 0, 0)),
            pl.BlockSpec((C1, C0), lambda b, t: (0, 0)),
            pl.BlockSpec((C1,), lambda b, t: (0,)),
            pl.BlockSpec((C0,), lambda b, t: (0,)),
            pl.BlockSpec((C0,), lambda b, t: (0,)),
        ],
        out_specs=[
            pl.BlockSpec((1, RT, C1), lambda b, t: (b, t, 0)),
            pl.BlockSpec((2, C1), lambda b, t: (0, 0)),
        ],
        out_shape=[
            jax.ShapeDtypeStruct((BN, NN * KS, C1), jnp.bfloat16),
            jax.ShapeDtypeStruct((2, C1), f32),
        ],
    )(ggath, rp, st0, tv1, cw1, cb1, g0, b0)

    y2, st2 = pl.pallas_call(
        _p3_body,
        grid=(BN, NT),
        in_specs=[
            pl.BlockSpec((1, RT, C1), lambda b, t: (b, t, 0)),
            pl.BlockSpec((2, C1), lambda b, t: (0, 0)),
            pl.BlockSpec((1, 8, C1), lambda b, t: (b, 0, 0)),
            pl.BlockSpec((C2, C1), lambda b, t: (0, 0)),
            pl.BlockSpec((C2,), lambda b, t: (0,)),
            pl.BlockSpec((C1,), lambda b, t: (0,)),
            pl.BlockSpec((C1,), lambda b, t: (0,)),
        ],
        out_specs=[
            pl.BlockSpec((1, RT, C2), lambda b, t: (b, t, 0)),
            pl.BlockSpec((2, C2), lambda b, t: (0, 0)),
        ],
        out_shape=[
            jax.ShapeDtypeStruct((BN, NN * KS, C2), jnp.bfloat16),
            jax.ShapeDtypeStruct((2, C2), f32),
        ],
    )(y1, st1, tv2, cw2, cb2, g1, b1)

    out = pl.pallas_call(
        _p4_body,
        grid=(BN, NT),
        in_specs=[
            pl.BlockSpec((1, RT, C2), lambda b, t: (b, t, 0)),
            pl.BlockSpec((2, C2), lambda b, t: (0, 0)),
            pl.BlockSpec((C2,), lambda b, t: (0,)),
            pl.BlockSpec((C2,), lambda b, t: (0,)),
        ],
        out_specs=pl.BlockSpec((1, C2, RT // KS), lambda b, t: (b, 0, t)),
        out_shape=jax.ShapeDtypeStruct((BN, C2, NN), f32),
    )(y2, st2, g2, b2)

    return out


# speedup vs baseline: 15.7801x; 1.0102x over previous
import functools

import jax
import jax.numpy as jnp
import numpy as np
from jax import lax
from jax.experimental import pallas as pl
from jax.experimental.pallas import tpu as pltpu
from jax.experimental.pallas import tpu_sc as plsc

BN = 2
NN = 2048
KS = 32
C0 = 64
C1 = 128
C2 = 256
R2 = np.float32(0.2 ** 2)
BIGI = np.int32(100000)
EPS = np.float32(1e-5)
CNTF = np.float32(BN * NN * KS)

TS = 256
SB = 8
RT = 4096
NT = (NN * KS) // RT


def _gelu(x):
    return x * (np.float32(0.5) * (np.float32(1.0) + lax.erf(
        x * np.float32(1.0 / np.sqrt(2.0)))))


def _k0_body(xyz_ref, ptsT_ref, temb_ref, tw0_ref, tb0_ref, cw0_ref,
             cb0_ref, tw1_ref, tb1_ref, tw2_ref, tb2_ref,
             g_ref, rp_ref, t1_ref, t2_ref):
    xyz = xyz_ref[0]
    ptsT = ptsT_ref[0]
    te = temb_ref[0]
    ge = _gelu(te)
    dn = (((1,), (1,)), ((), ()))
    t0 = lax.dot_general(ge, tw0_ref[...], dn,
                         preferred_element_type=jnp.float32) + tb0_ref[...]
    cw0 = cw0_ref[...]
    cw0x = cw0[:, 0:3]
    cw0p = cw0[:, 3:131]
    gx = lax.dot_general(xyz, cw0x, dn, preferred_element_type=jnp.float32)
    gp = lax.dot_general(ptsT, cw0p, dn, preferred_element_type=jnp.float32)
    gt = lax.dot_general(t0, cw0, dn, preferred_element_type=jnp.float32)
    gval = gx + gp + gt
    g_ref[0] = jnp.concatenate(
        [gval, jnp.zeros((NN, C0), jnp.float32)], axis=1)
    rp_ref[0] = gx - cb0_ref[...]
    t1 = lax.dot_general(ge, tw1_ref[...], dn,
                         preferred_element_type=jnp.float32) + tb1_ref[...]
    t2 = lax.dot_general(ge, tw2_ref[...], dn,
                         preferred_element_type=jnp.float32) + tb2_ref[...]
    t1_ref[0] = jnp.broadcast_to(t1, (8, C0))
    t2_ref[0] = jnp.broadcast_to(t2, (8, C1))


def _kmask_body(q_ref, xT_ref, rankp_ref):
    q = q_ref[0]
    xT = xT_ref[0]
    dx = q[:, 0:1] - xT[0:1, :]
    dy = q[:, 1:2] - xT[1:2, :]
    dz = q[:, 2:3] - xT[2:3, :]
    d2 = dx * dx + dy * dy + dz * dz
    mask = d2 <= R2
    mf = mask.astype(jnp.float32)
    r_in = lax.broadcasted_iota(jnp.int32, (128, 128), 0)
    c_in = lax.broadcasted_iota(jnp.int32, (128, 128), 1)
    tri_incl = (r_in <= c_in).astype(jnp.float32)
    mc = mf.reshape(TS * 16, 128)
    rank_in = jnp.dot(mc, tri_incl, preferred_element_type=jnp.float32)
    tot_c = mf.reshape(TS, 16, 128).sum(axis=2)
    r16 = lax.broadcasted_iota(jnp.int32, (16, 16), 0)
    c16 = lax.broadcasted_iota(jnp.int32, (16, 16), 1)
    tri_excl = (r16 < c16).astype(jnp.float32)
    pre = jnp.dot(tot_c, tri_excl, preferred_element_type=jnp.float32)
    rank = (rank_in.reshape(TS, 16, 128) + pre[:, :, None]).reshape(TS, NN)
    ranki = rank.astype(jnp.int32)
    sel = mask & (ranki <= KS)
    rankp_ref[...] = jnp.where(sel, ranki - 1, BIGI).reshape(TS * 16, 128)


NWORK = 32
RPW = (BN * NN) // NWORK
GR = 8
NG = RPW // GR
CPR = NN // 16
VR = NN // 128


def _sc_gather_body(rankp_hbm, gtab_hbm, out_hbm, rbuf, gi_ref, idx_ref,
                    rows_ref, cbuf, semr0, semr1, semg, semo0, semo1):
    cid = lax.axis_index("c")
    sid = lax.axis_index("s")
    wid = sid * 2 + cid
    row0 = wid * RPW
    boff = (wid // 16) * NN
    IL = GR * KS
    HP = IL // 2

    def issue_rank(g, h, sem):
        pltpu.async_copy(
            rankp_hbm.at[pl.ds((row0 + g * GR) * VR, GR * VR)],
            rbuf.at[pl.ds(h * GR * VR, GR * VR)], sem)

    def wait_rank(h, sem):
        pltpu.make_async_copy(
            rankp_hbm.at[pl.ds(0, GR * VR)],
            rbuf.at[pl.ds(h * GR * VR, GR * VR)], sem).wait()

    def issue_gather(h):
        for q in range(IL // 128):
            pltpu.async_copy(
                gtab_hbm.at[idx_ref.at[pl.ds(h * IL + q * 128, 128)]],
                rows_ref.at[pl.ds(q * 128, 128)], semg)

    def wait_gather():
        for q in range(IL // 128):
            pltpu.make_async_copy(
                gtab_hbm.at[idx_ref.at[pl.ds(q * 128, 128)]],
                rows_ref.at[pl.ds(q * 128, 128)], semg).wait()

    def pack(h):
        hc = h * HP

        @plsc.parallel_loop(0, HP, 1, unroll=4)
        def _(j):
            for v in range(C0 // 16):
                cbuf[hc + j, pl.ds(v * 16, 16)] = \
                    rows_ref[2 * j, pl.ds(v * 16, 16)]
                cbuf[hc + j, pl.ds(C0 + v * 16, 16)] = \
                    rows_ref[2 * j + 1, pl.ds(v * 16, 16)]

    def issue_out(g, h, sem):
        pltpu.async_copy(cbuf.at[pl.ds(h * HP, HP)],
                         out_hbm.at[pl.ds((row0 + g * GR) * (KS // 2), HP)],
                         sem)

    def wait_out(h, sem):
        pltpu.make_async_copy(cbuf.at[pl.ds(h * HP, HP)],
                              out_hbm.at[pl.ds(0, HP)], sem).wait()

    def compact(h):
        base_r = h * GR * VR
        neg = jnp.full((16,), -1, jnp.int32)
        for j in range(IL // 16):
            gi_ref[pl.ds(j * 16, 16)] = neg

        @plsc.parallel_loop(0, GR * CPR, 1, unroll=8)
        def _(i):
            r = i // CPR
            c = i % CPR
            rv = rbuf[base_r + r * VR + c // 8, pl.ds((c % 8) * 16, 16)]
            m = rv < KS
            slot = rv + r * KS
            nv = lax.iota(jnp.int32, 16) + c * 16
            plsc.store_scatter(gi_ref, [slot], nv, mask=m)

        for r in range(GR):
            v0 = gi_ref[pl.ds(r * KS, 16)]
            v1 = gi_ref[pl.ds(r * KS + 16, 16)]
            m0 = v0 >= 0
            m1 = v1 >= 0
            c0 = jnp.where(m0, v0, BIGI)
            c1 = jnp.where(m1, v1, BIGI)
            mn = jnp.minimum(jnp.min(c0), jnp.min(c1))
            idx_ref[pl.ds(h * IL + r * KS, 16)] = jnp.where(m0, v0, mn) + boff
            idx_ref[pl.ds(h * IL + r * KS + 16, 16)] = \
                jnp.where(m1, v1, mn) + boff

    issue_rank(0, 0, semr0)
    issue_rank(1, 1, semr1)

    def body(it, carry):
        g0 = it * 2
        g1 = g0 + 1
        wait_rank(0, semr0)
        compact(0)

        @pl.when(it > 0)
        def _():
            wait_gather()

            @pl.when(it > 1)
            def _():
                wait_out(1, semo1)

            pack(1)
            issue_out(g0 - 1, 1, semo1)

        issue_gather(0)

        @pl.when(g0 + 2 < NG)
        def _():
            issue_rank(g0 + 2, 0, semr0)

        wait_rank(1, semr1)
        compact(1)
        wait_gather()

        @pl.when(it > 0)
        def _():
            wait_out(0, semo0)

        pack(0)
        issue_out(g0, 0, semo0)
        issue_gather(1)

        @pl.when(g1 + 2 < NG)
        def _():
            issue_rank(g1 + 2, 1, semr1)
        return carry

    lax.fori_loop(0, NG // 2, body, 0)
    wait_gather()
    wait_out(1, semo1)
    pack(1)
    issue_out(NG - 1, 1, semo1)
    wait_out(1, semo1)
    wait_out(0, semo0)


def _sc_call(rankp, gtab):
    mesh = plsc.VectorSubcoreMesh(core_axis_name="c", subcore_axis_name="s")
    return pl.kernel(
        _sc_gather_body,
        out_type=jax.ShapeDtypeStruct((BN * NN * KS // 2, 2 * C0),
                                      jnp.float32),
        mesh=mesh,
        compiler_params=pltpu.CompilerParams(needs_layout_passes=False),
        scratch_types=[
            pltpu.VMEM((2 * GR * VR, 128), jnp.int32),
            pltpu.VMEM((GR * KS,), jnp.int32),
            pltpu.VMEM((2 * GR * KS,), jnp.int32),
            pltpu.VMEM((GR * KS, 2 * C0), jnp.float32),
            pltpu.VMEM((GR * KS, 2 * C0), jnp.float32),
            pltpu.SemaphoreType.DMA,
            pltpu.SemaphoreType.DMA,
            pltpu.SemaphoreType.DMA,
            pltpu.SemaphoreType.DMA,
            pltpu.SemaphoreType.DMA,
        ],
    )(rankp, gtab)


def _p1_body(gg_ref, rp_ref, st0_ref):
    gg = gg_ref[0]
    r = rp_ref[0]
    y0e = gg[:, 0:C0].reshape(RT // KS, KS // 2, C0) - r[:, None, :]
    y0o = gg[:, C0:2 * C0].reshape(RT // KS, KS // 2, C0) - r[:, None, :]
    s = jnp.sum(y0e, axis=(0, 1)) + jnp.sum(y0o, axis=(0, 1))
    q = jnp.sum(y0e * y0e, axis=(0, 1)) + jnp.sum(y0o * y0o, axis=(0, 1))

    @pl.when((pl.program_id(0) == 0) & (pl.program_id(1) == 0))
    def _():
        st0_ref[...] = jnp.zeros_like(st0_ref)

    st0_ref[...] += jnp.concatenate([s[None, :], q[None, :]], axis=0)


def _bn_coefs(st_ref, g_w, b_w):
    s = st_ref[0, :]
    q = st_ref[1, :]
    mean = s / CNTF
    var = q / CNTF - mean * mean
    scale = g_w * lax.rsqrt(var + EPS)
    shift = b_w - mean * scale
    return scale, shift


def _p2_body(gg_ref, rp_ref, st0_ref, tv1_ref, cw1_ref, cb1_ref,
             g0_ref, b0_ref, y1_ref, st1_ref):
    scale0, shift0 = _bn_coefs(st0_ref, g0_ref[...], b0_ref[...])
    gg = gg_ref[0]
    r = rp_ref[0]
    y0e = (gg[:, 0:C0].reshape(RT // KS, KS // 2, C0)
           - r[:, None, :]).reshape(RT // 2, C0)
    y0o = (gg[:, C0:2 * C0].reshape(RT // KS, KS // 2, C0)
           - r[:, None, :]).reshape(RT // 2, C0)
    tv = tv1_ref[0][0:1, :]
    ae = _gelu(y0e * scale0 + shift0) + tv
    ao = _gelu(y0o * scale0 + shift0) + tv
    dn = (((1,), (1,)), ((), ()))
    y1e = lax.dot_general(ae, cw1_ref[...], dn,
                          preferred_element_type=jnp.float32) + cb1_ref[...]
    y1o = lax.dot_general(ao, cw1_ref[...], dn,
                          preferred_element_type=jnp.float32) + cb1_ref[...]
    y1b = jnp.concatenate([y1e, y1o], axis=1).astype(jnp.bfloat16)
    y1_ref[0] = y1b
    y1s = y1b.astype(jnp.float32)
    y1a = y1s[:, 0:C1]
    y1c = y1s[:, C1:2 * C1]

    @pl.when((pl.program_id(0) == 0) & (pl.program_id(1) == 0))
    def _():
        st1_ref[...] = jnp.zeros_like(st1_ref)

    st1_ref[...] += jnp.concatenate(
        [(jnp.sum(y1a, axis=0) + jnp.sum(y1c, axis=0))[None, :],
         (jnp.sum(y1a * y1a, axis=0) + jnp.sum(y1c * y1c, axis=0))[None, :]],
        axis=0)


def _p3_body(y1_ref, st1_ref, tv2_ref, cw2_ref, cb2_ref,
             g1_ref, b1_ref, y2_ref, st2_ref):
    scale1, shift1 = _bn_coefs(st1_ref, g1_ref[...], b1_ref[...])
    y1 = y1_ref[0].astype(jnp.float32)
    tv = tv2_ref[0][0:1, :]
    ae = _gelu(y1[:, 0:C1] * scale1 + shift1) + tv
    ao = _gelu(y1[:, C1:2 * C1] * scale1 + shift1) + tv
    dn = (((1,), (1,)), ((), ()))
    y2e = lax.dot_general(ae, cw2_ref[...], dn,
                          preferred_element_type=jnp.float32) + cb2_ref[...]
    y2o = lax.dot_general(ao, cw2_ref[...], dn,
                          preferred_element_type=jnp.float32) + cb2_ref[...]
    y2b = jnp.concatenate([y2e, y2o], axis=1).astype(jnp.bfloat16)
    y2_ref[0] = y2b
    y2s = y2b.astype(jnp.float32)
    y2a = y2s[:, 0:C2]
    y2c = y2s[:, C2:2 * C2]

    @pl.when((pl.program_id(0) == 0) & (pl.program_id(1) == 0))
    def _():
        st2_ref[...] = jnp.zeros_like(st2_ref)

    st2_ref[...] += jnp.concatenate(
        [(jnp.sum(y2a, axis=0) + jnp.sum(y2c, axis=0))[None, :],
         (jnp.sum(y2a * y2a, axis=0) + jnp.sum(y2c * y2c, axis=0))[None, :]],
        axis=0)


def _p4_body(y2_ref, st2_ref, g2_ref, b2_ref, out_ref):
    scale2, shift2 = _bn_coefs(st2_ref, g2_ref[...], b2_ref[...])
    y2 = y2_ref[0].astype(jnp.float32)
    ze = (y2[:, 0:C2] * scale2 + shift2).reshape(RT // KS, KS // 2, C2)
    zo = (y2[:, C2:2 * C2] * scale2 + shift2).reshape(RT // KS, KS // 2, C2)
    zmx = jnp.maximum(jnp.max(ze, axis=1), jnp.max(zo, axis=1))
    zmn = jnp.minimum(jnp.min(ze, axis=1), jnp.min(zo, axis=1))
    m = jnp.maximum(_gelu(zmx), _gelu(zmn))
    out_ref[0] = m.T


def kernel(xyz, points, t_embed, tw0, tb0, cw0, cb0, g0, b0,
           tw1, tb1, cw1, cb1, g1, b1, tw2, tb2, cw2, cb2, g2, b2):
    f32 = jnp.float32
    ptsT = jnp.transpose(points, (0, 2, 1))
    xyzT = jnp.transpose(xyz, (0, 2, 1))
    temb3 = t_embed[:, None, :]

    g_tab, rp, tv1, tv2 = pl.pallas_call(
        _k0_body,
        grid=(BN,),
        in_specs=[
            pl.BlockSpec((1, NN, 3), lambda b: (b, 0, 0)),
            pl.BlockSpec((1, NN, 128), lambda b: (b, 0, 0)),
            pl.BlockSpec((1, 1, 256), lambda b: (b, 0, 0)),
            pl.BlockSpec((131, 256), lambda b: (0, 0)),
            pl.BlockSpec((131,), lambda b: (0,)),
            pl.BlockSpec((64, 131), lambda b: (0, 0)),
            pl.BlockSpec((64,), lambda b: (0,)),
            pl.BlockSpec((64, 256), lambda b: (0, 0)),
            pl.BlockSpec((64,), lambda b: (0,)),
            pl.BlockSpec((128, 256), lambda b: (0, 0)),
            pl.BlockSpec((128,), lambda b: (0,)),
        ],
        out_specs=[
            pl.BlockSpec((1, NN, 2 * C0), lambda b: (b, 0, 0)),
            pl.BlockSpec((1, NN, C0), lambda b: (b, 0, 0)),
            pl.BlockSpec((1, 8, C0), lambda b: (b, 0, 0)),
            pl.BlockSpec((1, 8, C1), lambda b: (b, 0, 0)),
        ],
        out_shape=[
            jax.ShapeDtypeStruct((BN, NN, 2 * C0), f32),
            jax.ShapeDtypeStruct((BN, NN, C0), f32),
            jax.ShapeDtypeStruct((BN, 8, C0), f32),
            jax.ShapeDtypeStruct((BN, 8, C1), f32),
        ],
    )(xyz, ptsT, temb3, tw0, tb0, cw0, cb0, tw1, tb1, tw2, tb2)

    rankp = pl.pallas_call(
        _kmask_body,
        grid=(BN, NN // TS),
        in_specs=[
            pl.BlockSpec((1, TS, 3), lambda b, j: (b, j, 0)),
            pl.BlockSpec((1, 3, NN), lambda b, j: (b, 0, 0)),
        ],
        out_specs=pl.BlockSpec((TS * 16, 128), lambda b, j: (b * 8 + j, 0)),
        out_shape=jax.ShapeDtypeStruct((BN * NN * 16, 128), jnp.int32),
    )(xyz, xyzT)

    ggath_flat = _sc_call(rankp, g_tab.reshape(BN * NN, 2 * C0))
    ggath = ggath_flat.reshape(BN, NN * KS // 2, 2 * C0)

    st0 = pl.pallas_call(
        _p1_body,
        grid=(BN, NT),
        in_specs=[
            pl.BlockSpec((1, RT // 2, 2 * C0), lambda b, t: (b, t, 0)),
            pl.BlockSpec((1, RT // KS, C0), lambda b, t: (b, t, 0)),
        ],
        out_specs=pl.BlockSpec((2, C0), lambda b, t: (0, 0)),
        out_shape=jax.ShapeDtypeStruct((2, C0), f32),
    )(ggath, rp)

    y1, st1 = pl.pallas_call(
        _p2_body,
        grid=(BN, NT),
        in_specs=[
            pl.BlockSpec((1, RT // 2, 2 * C0), lambda b, t: (b, t, 0)),
            pl.BlockSpec((1, RT // KS, C0), lambda b, t: (b, t, 0)),
            pl.BlockSpec((2, C0), lambda b, t: (0, 0)),
            pl.BlockSpec((1, 8, C0), lambda b, t: (b, 0, 0)),
            pl.BlockSpec((C1, C0), lambda b, t: (0, 0)),
            pl.BlockSpec((C1,), lambda b, t: (0,)),
            pl.BlockSpec((C0,), lambda b, t: (0,)),
            pl.BlockSpec((C0,), lambda b, t: (0,)),
        ],
        out_specs=[
            pl.BlockSpec((1, RT // 2, 2 * C1), lambda b, t: (b, t, 0)),
            pl.BlockSpec((2, C1), lambda b, t: (0, 0)),
        ],
        out_shape=[
            jax.ShapeDtypeStruct((BN, NN * KS // 2, 2 * C1), jnp.bfloat16),
            jax.ShapeDtypeStruct((2, C1), f32),
        ],
    )(ggath, rp, st0, tv1, cw1, cb1, g0, b0)

    y2, st2 = pl.pallas_call(
        _p3_body,
        grid=(BN, NT),
        in_specs=[
            pl.BlockSpec((1, RT // 2, 2 * C1), lambda b, t: (b, t, 0)),
            pl.BlockSpec((2, C1), lambda b, t: (0, 0)),
            pl.BlockSpec((1, 8, C1), lambda b, t: (b, 0, 0)),
            pl.BlockSpec((C2, C1), lambda b, t: (0, 0)),
            pl.BlockSpec((C2,), lambda b, t: (0,)),
            pl.BlockSpec((C1,), lambda b, t: (0,)),
            pl.BlockSpec((C1,), lambda b, t: (0,)),
        ],
        out_specs=[
            pl.BlockSpec((1, RT // 2, 2 * C2), lambda b, t: (b, t, 0)),
            pl.BlockSpec((2, C2), lambda b, t: (0, 0)),
        ],
        out_shape=[
            jax.ShapeDtypeStruct((BN, NN * KS // 2, 2 * C2), jnp.bfloat16),
            jax.ShapeDtypeStruct((2, C2), f32),
        ],
    )(y1, st1, tv2, cw2, cb2, g1, b1)

    out = pl.pallas_call(
        _p4_body,
        grid=(BN, NT),
        in_specs=[
            pl.BlockSpec((1, RT // 2, 2 * C2), lambda b, t: (b, t, 0)),
            pl.BlockSpec((2, C2), lambda b, t: (0, 0)),
            pl.BlockSpec((C2,), lambda b, t: (0,)),
            pl.BlockSpec((C2,), lambda b, t: (0,)),
        ],
        out_specs=pl.BlockSpec((1, C2, RT // KS), lambda b, t: (b, 0, t)),
        out_shape=jax.ShapeDtypeStruct((BN, C2, NN), f32),
    )(y2, st2, g2, b2)

    return out


# speedup vs baseline: 16.7176x; 1.0594x over previous
import functools

import jax
import jax.numpy as jnp
import numpy as np
from jax import lax
from jax.experimental import pallas as pl
from jax.experimental.pallas import tpu as pltpu
from jax.experimental.pallas import tpu_sc as plsc

BN = 2
NN = 2048
KS = 32
C0 = 64
C1 = 128
C2 = 256
R2 = np.float32(0.2 ** 2)
BIGI = np.int32(100000)
EPS = np.float32(1e-5)
CNTF = np.float32(BN * NN * KS)

TS = 256
SB = 8
RT = 8192
NT = (NN * KS) // RT


def _gelu(x):
    return x * (np.float32(0.5) * (np.float32(1.0) + lax.erf(
        x * np.float32(1.0 / np.sqrt(2.0)))))


def _k0_body(xyz_ref, ptsT_ref, temb_ref, tw0_ref, tb0_ref, cw0_ref,
             cb0_ref, tw1_ref, tb1_ref, tw2_ref, tb2_ref,
             g_ref, rp_ref, t1_ref, t2_ref):
    xyz = xyz_ref[0]
    ptsT = ptsT_ref[0]
    te = temb_ref[0]
    ge = _gelu(te)
    dn = (((1,), (1,)), ((), ()))
    t0 = lax.dot_general(ge, tw0_ref[...], dn,
                         preferred_element_type=jnp.float32) + tb0_ref[...]
    cw0 = cw0_ref[...]
    cw0x = cw0[:, 0:3]
    cw0p = cw0[:, 3:131]
    gx = lax.dot_general(xyz, cw0x, dn, preferred_element_type=jnp.float32)
    gp = lax.dot_general(ptsT, cw0p, dn, preferred_element_type=jnp.float32)
    gt = lax.dot_general(t0, cw0, dn, preferred_element_type=jnp.float32)
    gval = gx + gp + gt
    g_ref[0] = jnp.concatenate(
        [gval, jnp.zeros((NN, C0), jnp.float32)], axis=1)
    rp_ref[0] = gx - cb0_ref[...]
    t1 = lax.dot_general(ge, tw1_ref[...], dn,
                         preferred_element_type=jnp.float32) + tb1_ref[...]
    t2 = lax.dot_general(ge, tw2_ref[...], dn,
                         preferred_element_type=jnp.float32) + tb2_ref[...]
    t1_ref[0] = jnp.broadcast_to(t1, (8, C0))
    t2_ref[0] = jnp.broadcast_to(t2, (8, C1))


def _kmask_body(q_ref, xT_ref, rankp_ref):
    q = q_ref[0]
    xT = xT_ref[0]
    dx = q[:, 0:1] - xT[0:1, :]
    dy = q[:, 1:2] - xT[1:2, :]
    dz = q[:, 2:3] - xT[2:3, :]
    d2 = dx * dx + dy * dy + dz * dz
    mask = d2 <= R2
    mf = mask.astype(jnp.float32)
    r_in = lax.broadcasted_iota(jnp.int32, (128, 128), 0)
    c_in = lax.broadcasted_iota(jnp.int32, (128, 128), 1)
    tri_incl = (r_in <= c_in).astype(jnp.float32)
    mc = mf.reshape(TS * 16, 128)
    rank_in = jnp.dot(mc, tri_incl, preferred_element_type=jnp.float32)
    tot_c = mf.reshape(TS, 16, 128).sum(axis=2)
    r16 = lax.broadcasted_iota(jnp.int32, (16, 16), 0)
    c16 = lax.broadcasted_iota(jnp.int32, (16, 16), 1)
    tri_excl = (r16 < c16).astype(jnp.float32)
    pre = jnp.dot(tot_c, tri_excl, preferred_element_type=jnp.float32)
    rank = (rank_in.reshape(TS, 16, 128) + pre[:, :, None]).reshape(TS, NN)
    ranki = rank.astype(jnp.int32)
    sel = mask & (ranki <= KS)
    rankp_ref[...] = jnp.where(sel, ranki - 1, BIGI).reshape(TS * 16, 128)


NWORK = 32
RPW = (BN * NN) // NWORK
GR = 8
NG = RPW // GR
CPR = NN // 16
VR = NN // 128


def _sc_gather_body(rankp_hbm, gtab_hbm, out_hbm, rbuf, gi_ref, idx_ref,
                    rows_ref, cbuf, semr0, semr1, semg, semo0, semo1):
    cid = lax.axis_index("c")
    sid = lax.axis_index("s")
    wid = sid * 2 + cid
    row0 = wid * RPW
    boff = (wid // 16) * NN
    IL = GR * KS
    HP = IL // 2

    def issue_rank(g, h, sem):
        pltpu.async_copy(
            rankp_hbm.at[pl.ds((row0 + g * GR) * VR, GR * VR)],
            rbuf.at[pl.ds(h * GR * VR, GR * VR)], sem)

    def wait_rank(h, sem):
        pltpu.make_async_copy(
            rankp_hbm.at[pl.ds(0, GR * VR)],
            rbuf.at[pl.ds(h * GR * VR, GR * VR)], sem).wait()

    def issue_gather(h):
        for q in range(IL // 128):
            pltpu.async_copy(
                gtab_hbm.at[idx_ref.at[pl.ds(h * IL + q * 128, 128)]],
                rows_ref.at[pl.ds(q * 128, 128)], semg)

    def wait_gather():
        for q in range(IL // 128):
            pltpu.make_async_copy(
                gtab_hbm.at[idx_ref.at[pl.ds(q * 128, 128)]],
                rows_ref.at[pl.ds(q * 128, 128)], semg).wait()

    def pack(h):
        hc = h * HP

        @plsc.parallel_loop(0, HP, 1, unroll=4)
        def _(j):
            for v in range(C0 // 16):
                cbuf[hc + j, pl.ds(v * 16, 16)] = \
                    rows_ref[2 * j, pl.ds(v * 16, 16)]
                cbuf[hc + j, pl.ds(C0 + v * 16, 16)] = \
                    rows_ref[2 * j + 1, pl.ds(v * 16, 16)]

    def issue_out(g, h, sem):
        pltpu.async_copy(cbuf.at[pl.ds(h * HP, HP)],
                         out_hbm.at[pl.ds((row0 + g * GR) * (KS // 2), HP)],
                         sem)

    def wait_out(h, sem):
        pltpu.make_async_copy(cbuf.at[pl.ds(h * HP, HP)],
                              out_hbm.at[pl.ds(0, HP)], sem).wait()

    def compact(h):
        base_r = h * GR * VR
        neg = jnp.full((16,), -1, jnp.int32)
        for j in range(IL // 16):
            gi_ref[pl.ds(j * 16, 16)] = neg

        @plsc.parallel_loop(0, GR * CPR, 1, unroll=8)
        def _(i):
            r = i // CPR
            c = i % CPR
            rv = rbuf[base_r + r * VR + c // 8, pl.ds((c % 8) * 16, 16)]
            m = rv < KS
            slot = rv + r * KS
            nv = lax.iota(jnp.int32, 16) + c * 16
            plsc.store_scatter(gi_ref, [slot], nv, mask=m)

        for r in range(GR):
            v0 = gi_ref[pl.ds(r * KS, 16)]
            v1 = gi_ref[pl.ds(r * KS + 16, 16)]
            m0 = v0 >= 0
            m1 = v1 >= 0
            c0 = jnp.where(m0, v0, BIGI)
            c1 = jnp.where(m1, v1, BIGI)
            mn = jnp.minimum(jnp.min(c0), jnp.min(c1))
            idx_ref[pl.ds(h * IL + r * KS, 16)] = jnp.where(m0, v0, mn) + boff
            idx_ref[pl.ds(h * IL + r * KS + 16, 16)] = \
                jnp.where(m1, v1, mn) + boff

    issue_rank(0, 0, semr0)
    issue_rank(1, 1, semr1)

    def body(it, carry):
        g0 = it * 2
        g1 = g0 + 1
        wait_rank(0, semr0)
        compact(0)

        @pl.when(it > 0)
        def _():
            wait_gather()

            @pl.when(it > 1)
            def _():
                wait_out(1, semo1)

            pack(1)
            issue_out(g0 - 1, 1, semo1)

        issue_gather(0)

        @pl.when(g0 + 2 < NG)
        def _():
            issue_rank(g0 + 2, 0, semr0)

        wait_rank(1, semr1)
        compact(1)
        wait_gather()

        @pl.when(it > 0)
        def _():
            wait_out(0, semo0)

        pack(0)
        issue_out(g0, 0, semo0)
        issue_gather(1)

        @pl.when(g1 + 2 < NG)
        def _():
            issue_rank(g1 + 2, 1, semr1)
        return carry

    lax.fori_loop(0, NG // 2, body, 0)
    wait_gather()
    wait_out(1, semo1)
    pack(1)
    issue_out(NG - 1, 1, semo1)
    wait_out(1, semo1)
    wait_out(0, semo0)


def _sc_call(rankp, gtab):
    mesh = plsc.VectorSubcoreMesh(core_axis_name="c", subcore_axis_name="s")
    return pl.kernel(
        _sc_gather_body,
        out_type=jax.ShapeDtypeStruct((BN * NN * KS // 2, 2 * C0),
                                      jnp.float32),
        mesh=mesh,
        compiler_params=pltpu.CompilerParams(needs_layout_passes=False),
        scratch_types=[
            pltpu.VMEM((2 * GR * VR, 128), jnp.int32),
            pltpu.VMEM((GR * KS,), jnp.int32),
            pltpu.VMEM((2 * GR * KS,), jnp.int32),
            pltpu.VMEM((GR * KS, 2 * C0), jnp.float32),
            pltpu.VMEM((GR * KS, 2 * C0), jnp.float32),
            pltpu.SemaphoreType.DMA,
            pltpu.SemaphoreType.DMA,
            pltpu.SemaphoreType.DMA,
            pltpu.SemaphoreType.DMA,
            pltpu.SemaphoreType.DMA,
        ],
    )(rankp, gtab)


def _p1_body(gg_ref, rp_ref, st0_ref):
    gg = gg_ref[0]
    r = rp_ref[0]
    y0e = gg[:, 0:C0].reshape(RT // KS, KS // 2, C0) - r[:, None, :]
    y0o = gg[:, C0:2 * C0].reshape(RT // KS, KS // 2, C0) - r[:, None, :]
    s = jnp.sum(y0e, axis=(0, 1)) + jnp.sum(y0o, axis=(0, 1))
    q = jnp.sum(y0e * y0e, axis=(0, 1)) + jnp.sum(y0o * y0o, axis=(0, 1))

    @pl.when((pl.program_id(0) == 0) & (pl.program_id(1) == 0))
    def _():
        st0_ref[...] = jnp.zeros_like(st0_ref)

    st0_ref[...] += jnp.concatenate([s[None, :], q[None, :]], axis=0)


def _bn_coefs(st_ref, g_w, b_w):
    s = st_ref[0, :]
    q = st_ref[1, :]
    mean = s / CNTF
    var = q / CNTF - mean * mean
    scale = g_w * lax.rsqrt(var + EPS)
    shift = b_w - mean * scale
    return scale, shift


def _p2_body(gg_ref, rp_ref, st0_ref, tv1_ref, cw1_ref, cb1_ref,
             g0_ref, b0_ref, y1_ref, st1_ref):
    scale0, shift0 = _bn_coefs(st0_ref, g0_ref[...], b0_ref[...])
    gg = gg_ref[0]
    r = rp_ref[0]
    y0e = (gg[:, 0:C0].reshape(RT // KS, KS // 2, C0)
           - r[:, None, :]).reshape(RT // 2, C0)
    y0o = (gg[:, C0:2 * C0].reshape(RT // KS, KS // 2, C0)
           - r[:, None, :]).reshape(RT // 2, C0)
    tv = tv1_ref[0][0:1, :]
    ae = _gelu(y0e * scale0 + shift0) + tv
    ao = _gelu(y0o * scale0 + shift0) + tv
    dn = (((1,), (1,)), ((), ()))
    y1e = lax.dot_general(ae, cw1_ref[...], dn,
                          preferred_element_type=jnp.float32) + cb1_ref[...]
    y1o = lax.dot_general(ao, cw1_ref[...], dn,
                          preferred_element_type=jnp.float32) + cb1_ref[...]
    y1b = jnp.concatenate([y1e, y1o], axis=1).astype(jnp.bfloat16)
    y1_ref[0] = y1b
    y1s = y1b.astype(jnp.float32)
    y1a = y1s[:, 0:C1]
    y1c = y1s[:, C1:2 * C1]

    @pl.when((pl.program_id(0) == 0) & (pl.program_id(1) == 0))
    def _():
        st1_ref[...] = jnp.zeros_like(st1_ref)

    st1_ref[...] += jnp.concatenate(
        [(jnp.sum(y1a, axis=0) + jnp.sum(y1c, axis=0))[None, :],
         (jnp.sum(y1a * y1a, axis=0) + jnp.sum(y1c * y1c, axis=0))[None, :]],
        axis=0)


def _p3_body(y1_ref, st1_ref, tv2_ref, cw2_ref, cb2_ref,
             g1_ref, b1_ref, y2_ref, st2_ref):
    scale1, shift1 = _bn_coefs(st1_ref, g1_ref[...], b1_ref[...])
    y1 = y1_ref[0].astype(jnp.float32)
    tv = tv2_ref[0][0:1, :]
    ae = _gelu(y1[:, 0:C1] * scale1 + shift1) + tv
    ao = _gelu(y1[:, C1:2 * C1] * scale1 + shift1) + tv
    dn = (((1,), (1,)), ((), ()))
    y2e = lax.dot_general(ae, cw2_ref[...], dn,
                          preferred_element_type=jnp.float32) + cb2_ref[...]
    y2o = lax.dot_general(ao, cw2_ref[...], dn,
                          preferred_element_type=jnp.float32) + cb2_ref[...]
    y2b = jnp.concatenate([y2e, y2o], axis=1).astype(jnp.bfloat16)
    y2_ref[0] = y2b
    y2s = y2b.astype(jnp.float32)
    y2a = y2s[:, 0:C2]
    y2c = y2s[:, C2:2 * C2]

    @pl.when((pl.program_id(0) == 0) & (pl.program_id(1) == 0))
    def _():
        st2_ref[...] = jnp.zeros_like(st2_ref)

    st2_ref[...] += jnp.concatenate(
        [(jnp.sum(y2a, axis=0) + jnp.sum(y2c, axis=0))[None, :],
         (jnp.sum(y2a * y2a, axis=0) + jnp.sum(y2c * y2c, axis=0))[None, :]],
        axis=0)


def _p4_body(y2_ref, st2_ref, g2_ref, b2_ref, out_ref):
    scale2, shift2 = _bn_coefs(st2_ref, g2_ref[...], b2_ref[...])
    y2 = y2_ref[0].astype(jnp.float32)
    ze = (y2[:, 0:C2] * scale2 + shift2).reshape(RT // KS, KS // 2, C2)
    zo = (y2[:, C2:2 * C2] * scale2 + shift2).reshape(RT // KS, KS // 2, C2)
    zmx = jnp.maximum(jnp.max(ze, axis=1), jnp.max(zo, axis=1))
    zmn = jnp.minimum(jnp.min(ze, axis=1), jnp.min(zo, axis=1))
    m = jnp.maximum(_gelu(zmx), _gelu(zmn))
    out_ref[0] = m.T


def kernel(xyz, points, t_embed, tw0, tb0, cw0, cb0, g0, b0,
           tw1, tb1, cw1, cb1, g1, b1, tw2, tb2, cw2, cb2, g2, b2):
    f32 = jnp.float32
    ptsT = jnp.transpose(points, (0, 2, 1))
    xyzT = jnp.transpose(xyz, (0, 2, 1))
    temb3 = t_embed[:, None, :]

    g_tab, rp, tv1, tv2 = pl.pallas_call(
        _k0_body,
        grid=(BN,),
        in_specs=[
            pl.BlockSpec((1, NN, 3), lambda b: (b, 0, 0)),
            pl.BlockSpec((1, NN, 128), lambda b: (b, 0, 0)),
            pl.BlockSpec((1, 1, 256), lambda b: (b, 0, 0)),
            pl.BlockSpec((131, 256), lambda b: (0, 0)),
            pl.BlockSpec((131,), lambda b: (0,)),
            pl.BlockSpec((64, 131), lambda b: (0, 0)),
            pl.BlockSpec((64,), lambda b: (0,)),
            pl.BlockSpec((64, 256), lambda b: (0, 0)),
            pl.BlockSpec((64,), lambda b: (0,)),
            pl.BlockSpec((128, 256), lambda b: (0, 0)),
            pl.BlockSpec((128,), lambda b: (0,)),
        ],
        out_specs=[
            pl.BlockSpec((1, NN, 2 * C0), lambda b: (b, 0, 0)),
            pl.BlockSpec((1, NN, C0), lambda b: (b, 0, 0)),
            pl.BlockSpec((1, 8, C0), lambda b: (b, 0, 0)),
            pl.BlockSpec((1, 8, C1), lambda b: (b, 0, 0)),
        ],
        out_shape=[
            jax.ShapeDtypeStruct((BN, NN, 2 * C0), f32),
            jax.ShapeDtypeStruct((BN, NN, C0), f32),
            jax.ShapeDtypeStruct((BN, 8, C0), f32),
            jax.ShapeDtypeStruct((BN, 8, C1), f32),
        ],
    )(xyz, ptsT, temb3, tw0, tb0, cw0, cb0, tw1, tb1, tw2, tb2)

    rankp = pl.pallas_call(
        _kmask_body,
        grid=(BN, NN // TS),
        in_specs=[
            pl.BlockSpec((1, TS, 3), lambda b, j: (b, j, 0)),
            pl.BlockSpec((1, 3, NN), lambda b, j: (b, 0, 0)),
        ],
        out_specs=pl.BlockSpec((TS * 16, 128), lambda b, j: (b * 8 + j, 0)),
        out_shape=jax.ShapeDtypeStruct((BN * NN * 16, 128), jnp.int32),
    )(xyz, xyzT)

    ggath_flat = _sc_call(rankp, g_tab.reshape(BN * NN, 2 * C0))
    ggath = ggath_flat.reshape(BN, NN * KS // 2, 2 * C0)

    st0 = pl.pallas_call(
        _p1_body,
        grid=(BN, NT),
        in_specs=[
            pl.BlockSpec((1, RT // 2, 2 * C0), lambda b, t: (b, t, 0)),
            pl.BlockSpec((1, RT // KS, C0), lambda b, t: (b, t, 0)),
        ],
        out_specs=pl.BlockSpec((2, C0), lambda b, t: (0, 0)),
        out_shape=jax.ShapeDtypeStruct((2, C0), f32),
    )(ggath, rp)

    y1, st1 = pl.pallas_call(
        _p2_body,
        grid=(BN, NT),
        in_specs=[
            pl.BlockSpec((1, RT // 2, 2 * C0), lambda b, t: (b, t, 0)),
            pl.BlockSpec((1, RT // KS, C0), lambda b, t: (b, t, 0)),
            pl.BlockSpec((2, C0), lambda b, t: (0, 0)),
            pl.BlockSpec((1, 8, C0), lambda b, t: (b, 0, 0)),
            pl.BlockSpec((C1, C0), lambda b, t: (0, 0)),
            pl.BlockSpec((C1,), lambda b, t: (0,)),
            pl.BlockSpec((C0,), lambda b, t: (0,)),
            pl.BlockSpec((C0,), lambda b, t: (0,)),
        ],
        out_specs=[
            pl.BlockSpec((1, RT // 2, 2 * C1), lambda b, t: (b, t, 0)),
            pl.BlockSpec((2, C1), lambda b, t: (0, 0)),
        ],
        out_shape=[
            jax.ShapeDtypeStruct((BN, NN * KS // 2, 2 * C1), jnp.bfloat16),
            jax.ShapeDtypeStruct((2, C1), f32),
        ],
    )(ggath, rp, st0, tv1, cw1, cb1, g0, b0)

    y2, st2 = pl.pallas_call(
        _p3_body,
        grid=(BN, NT),
        in_specs=[
            pl.BlockSpec((1, RT // 2, 2 * C1), lambda b, t: (b, t, 0)),
            pl.BlockSpec((2, C1), lambda b, t: (0, 0)),
            pl.BlockSpec((1, 8, C1), lambda b, t: (b, 0, 0)),
            pl.BlockSpec((C2, C1), lambda b, t: (0, 0)),
            pl.BlockSpec((C2,), lambda b, t: (0,)),
            pl.BlockSpec((C1,), lambda b, t: (0,)),
            pl.BlockSpec((C1,), lambda b, t: (0,)),
        ],
        out_specs=[
            pl.BlockSpec((1, RT // 2, 2 * C2), lambda b, t: (b, t, 0)),
            pl.BlockSpec((2, C2), lambda b, t: (0, 0)),
        ],
        out_shape=[
            jax.ShapeDtypeStruct((BN, NN * KS // 2, 2 * C2), jnp.bfloat16),
            jax.ShapeDtypeStruct((2, C2), f32),
        ],
    )(y1, st1, tv2, cw2, cb2, g1, b1)

    out = pl.pallas_call(
        _p4_body,
        grid=(BN, NT),
        in_specs=[
            pl.BlockSpec((1, RT // 2, 2 * C2), lambda b, t: (b, t, 0)),
            pl.BlockSpec((2, C2), lambda b, t: (0, 0)),
            pl.BlockSpec((C2,), lambda b, t: (0,)),
            pl.BlockSpec((C2,), lambda b, t: (0,)),
        ],
        out_specs=pl.BlockSpec((1, C2, RT // KS), lambda b, t: (b, 0, t)),
        out_shape=jax.ShapeDtypeStruct((BN, C2, NN), f32),
    )(y2, st2, g2, b2)

    return out


# speedup vs baseline: 16.7910x; 1.0044x over previous
import functools

import jax
import jax.numpy as jnp
import numpy as np
from jax import lax
from jax.experimental import pallas as pl
from jax.experimental.pallas import tpu as pltpu
from jax.experimental.pallas import tpu_sc as plsc

BN = 2
NN = 2048
KS = 32
C0 = 64
C1 = 128
C2 = 256
R2 = np.float32(0.2 ** 2)
BIGI = np.int32(100000)
EPS = np.float32(1e-5)
CNTF = np.float32(BN * NN * KS)

TS = 512
SB = 8
RT = 8192
NT = (NN * KS) // RT


def _gelu(x):
    return x * (np.float32(0.5) * (np.float32(1.0) + lax.erf(
        x * np.float32(1.0 / np.sqrt(2.0)))))


def _k0_body(xyz_ref, ptsT_ref, temb_ref, tw0_ref, tb0_ref, cw0_ref,
             cb0_ref, tw1_ref, tb1_ref, tw2_ref, tb2_ref,
             g_ref, rp_ref, t1_ref, t2_ref):
    xyz = xyz_ref[0]
    ptsT = ptsT_ref[0]
    te = temb_ref[0]
    ge = _gelu(te)
    dn = (((1,), (1,)), ((), ()))
    t0 = lax.dot_general(ge, tw0_ref[...], dn,
                         preferred_element_type=jnp.float32) + tb0_ref[...]
    cw0 = cw0_ref[...]
    cw0x = cw0[:, 0:3]
    cw0p = cw0[:, 3:131]
    gx = lax.dot_general(xyz, cw0x, dn, preferred_element_type=jnp.float32)
    gp = lax.dot_general(ptsT, cw0p, dn, preferred_element_type=jnp.float32)
    gt = lax.dot_general(t0, cw0, dn, preferred_element_type=jnp.float32)
    gval = gx + gp + gt
    g_ref[0] = jnp.concatenate(
        [gval, jnp.zeros((NN, C0), jnp.float32)], axis=1)
    rp_ref[0] = gx - cb0_ref[...]
    t1 = lax.dot_general(ge, tw1_ref[...], dn,
                         preferred_element_type=jnp.float32) + tb1_ref[...]
    t2 = lax.dot_general(ge, tw2_ref[...], dn,
                         preferred_element_type=jnp.float32) + tb2_ref[...]
    t1_ref[0] = jnp.broadcast_to(t1, (8, C0))
    t2_ref[0] = jnp.broadcast_to(t2, (8, C1))


def _kmask_body(q_ref, xT_ref, rankp_ref):
    q = q_ref[0]
    xT = xT_ref[0]
    dx = q[:, 0:1] - xT[0:1, :]
    dy = q[:, 1:2] - xT[1:2, :]
    dz = q[:, 2:3] - xT[2:3, :]
    d2 = dx * dx + dy * dy + dz * dz
    mask = d2 <= R2
    mf = mask.astype(jnp.float32)
    r_in = lax.broadcasted_iota(jnp.int32, (128, 128), 0)
    c_in = lax.broadcasted_iota(jnp.int32, (128, 128), 1)
    tri_incl = (r_in <= c_in).astype(jnp.float32)
    mc = mf.reshape(TS * 16, 128)
    rank_in = jnp.dot(mc, tri_incl, preferred_element_type=jnp.float32)
    tot_c = mf.reshape(TS, 16, 128).sum(axis=2)
    r16 = lax.broadcasted_iota(jnp.int32, (16, 16), 0)
    c16 = lax.broadcasted_iota(jnp.int32, (16, 16), 1)
    tri_excl = (r16 < c16).astype(jnp.float32)
    pre = jnp.dot(tot_c, tri_excl, preferred_element_type=jnp.float32)
    rank = (rank_in.reshape(TS, 16, 128) + pre[:, :, None]).reshape(TS, NN)
    ranki = rank.astype(jnp.int32)
    sel = mask & (ranki <= KS)
    rankp_ref[...] = jnp.where(sel, ranki - 1, BIGI).reshape(TS * 16, 128)


NWORK = 32
RPW = (BN * NN) // NWORK
GR = 8
NG = RPW // GR
CPR = NN // 16
VR = NN // 128


def _sc_gather_body(rankp_hbm, gtab_hbm, out_hbm, rbuf, gi_ref, idx_ref,
                    rows_ref, cbuf, semr0, semr1, semg, semo0, semo1):
    cid = lax.axis_index("c")
    sid = lax.axis_index("s")
    wid = sid * 2 + cid
    row0 = wid * RPW
    boff = (wid // 16) * NN
    IL = GR * KS
    HP = IL // 2

    def issue_rank(g, h, sem):
        pltpu.async_copy(
            rankp_hbm.at[pl.ds((row0 + g * GR) * VR, GR * VR)],
            rbuf.at[pl.ds(h * GR * VR, GR * VR)], sem)

    def wait_rank(h, sem):
        pltpu.make_async_copy(
            rankp_hbm.at[pl.ds(0, GR * VR)],
            rbuf.at[pl.ds(h * GR * VR, GR * VR)], sem).wait()

    def issue_gather(h):
        for q in range(IL // 128):
            pltpu.async_copy(
                gtab_hbm.at[idx_ref.at[pl.ds(h * IL + q * 128, 128)]],
                rows_ref.at[pl.ds(q * 128, 128)], semg)

    def wait_gather():
        for q in range(IL // 128):
            pltpu.make_async_copy(
                gtab_hbm.at[idx_ref.at[pl.ds(q * 128, 128)]],
                rows_ref.at[pl.ds(q * 128, 128)], semg).wait()

    def pack(h):
        hc = h * HP

        @plsc.parallel_loop(0, HP, 1, unroll=4)
        def _(j):
            for v in range(C0 // 16):
                cbuf[hc + j, pl.ds(v * 16, 16)] = \
                    rows_ref[2 * j, pl.ds(v * 16, 16)]
                cbuf[hc + j, pl.ds(C0 + v * 16, 16)] = \
                    rows_ref[2 * j + 1, pl.ds(v * 16, 16)]

    def issue_out(g, h, sem):
        pltpu.async_copy(cbuf.at[pl.ds(h * HP, HP)],
                         out_hbm.at[pl.ds((row0 + g * GR) * (KS // 2), HP)],
                         sem)

    def wait_out(h, sem):
        pltpu.make_async_copy(cbuf.at[pl.ds(h * HP, HP)],
                              out_hbm.at[pl.ds(0, HP)], sem).wait()

    def compact(h):
        base_r = h * GR * VR
        neg = jnp.full((16,), -1, jnp.int32)
        for j in range(IL // 16):
            gi_ref[pl.ds(j * 16, 16)] = neg

        @plsc.parallel_loop(0, GR * CPR, 1, unroll=8)
        def _(i):
            r = i // CPR
            c = i % CPR
            rv = rbuf[base_r + r * VR + c // 8, pl.ds((c % 8) * 16, 16)]
            m = rv < KS
            slot = rv + r * KS
            nv = lax.iota(jnp.int32, 16) + c * 16
            plsc.store_scatter(gi_ref, [slot], nv, mask=m)

        for r in range(GR):
            v0 = gi_ref[pl.ds(r * KS, 16)]
            v1 = gi_ref[pl.ds(r * KS + 16, 16)]
            m0 = v0 >= 0
            m1 = v1 >= 0
            c0 = jnp.where(m0, v0, BIGI)
            c1 = jnp.where(m1, v1, BIGI)
            mn = jnp.minimum(jnp.min(c0), jnp.min(c1))
            idx_ref[pl.ds(h * IL + r * KS, 16)] = jnp.where(m0, v0, mn) + boff
            idx_ref[pl.ds(h * IL + r * KS + 16, 16)] = \
                jnp.where(m1, v1, mn) + boff

    issue_rank(0, 0, semr0)
    issue_rank(1, 1, semr1)

    def body(it, carry):
        g0 = it * 2
        g1 = g0 + 1
        wait_rank(0, semr0)
        compact(0)

        @pl.when(it > 0)
        def _():
            wait_gather()

            @pl.when(it > 1)
            def _():
                wait_out(1, semo1)

            pack(1)
            issue_out(g0 - 1, 1, semo1)

        issue_gather(0)

        @pl.when(g0 + 2 < NG)
        def _():
            issue_rank(g0 + 2, 0, semr0)

        wait_rank(1, semr1)
        compact(1)
        wait_gather()

        @pl.when(it > 0)
        def _():
            wait_out(0, semo0)

        pack(0)
        issue_out(g0, 0, semo0)
        issue_gather(1)

        @pl.when(g1 + 2 < NG)
        def _():
            issue_rank(g1 + 2, 1, semr1)
        return carry

    lax.fori_loop(0, NG // 2, body, 0)
    wait_gather()
    wait_out(1, semo1)
    pack(1)
    issue_out(NG - 1, 1, semo1)
    wait_out(1, semo1)
    wait_out(0, semo0)


def _sc_call(rankp, gtab):
    mesh = plsc.VectorSubcoreMesh(core_axis_name="c", subcore_axis_name="s")
    return pl.kernel(
        _sc_gather_body,
        out_type=jax.ShapeDtypeStruct((BN * NN * KS // 2, 2 * C0),
                                      jnp.float32),
        mesh=mesh,
        compiler_params=pltpu.CompilerParams(needs_layout_passes=False),
        scratch_types=[
            pltpu.VMEM((2 * GR * VR, 128), jnp.int32),
            pltpu.VMEM((GR * KS,), jnp.int32),
            pltpu.VMEM((2 * GR * KS,), jnp.int32),
            pltpu.VMEM((GR * KS, 2 * C0), jnp.float32),
            pltpu.VMEM((GR * KS, 2 * C0), jnp.float32),
            pltpu.SemaphoreType.DMA,
            pltpu.SemaphoreType.DMA,
            pltpu.SemaphoreType.DMA,
            pltpu.SemaphoreType.DMA,
            pltpu.SemaphoreType.DMA,
        ],
    )(rankp, gtab)


def _p1_body(gg_ref, rp_ref, st0_ref):
    gg = gg_ref[0]
    r = rp_ref[0]
    y0e = gg[:, 0:C0].reshape(RT // KS, KS // 2, C0) - r[:, None, :]
    y0o = gg[:, C0:2 * C0].reshape(RT // KS, KS // 2, C0) - r[:, None, :]
    s = jnp.sum(y0e, axis=(0, 1)) + jnp.sum(y0o, axis=(0, 1))
    q = jnp.sum(y0e * y0e, axis=(0, 1)) + jnp.sum(y0o * y0o, axis=(0, 1))

    @pl.when((pl.program_id(0) == 0) & (pl.program_id(1) == 0))
    def _():
        st0_ref[...] = jnp.zeros_like(st0_ref)

    st0_ref[...] += jnp.concatenate([s[None, :], q[None, :]], axis=0)


def _bn_coefs(st_ref, g_w, b_w):
    s = st_ref[0, :]
    q = st_ref[1, :]
    mean = s / CNTF
    var = q / CNTF - mean * mean
    scale = g_w * lax.rsqrt(var + EPS)
    shift = b_w - mean * scale
    return scale, shift


def _p2_body(gg_ref, rp_ref, st0_ref, tv1_ref, cw1_ref, cb1_ref,
             g0_ref, b0_ref, y1_ref, st1_ref):
    scale0, shift0 = _bn_coefs(st0_ref, g0_ref[...], b0_ref[...])
    gg = gg_ref[0]
    r = rp_ref[0]
    y0e = (gg[:, 0:C0].reshape(RT // KS, KS // 2, C0)
           - r[:, None, :]).reshape(RT // 2, C0)
    y0o = (gg[:, C0:2 * C0].reshape(RT // KS, KS // 2, C0)
           - r[:, None, :]).reshape(RT // 2, C0)
    tv = tv1_ref[0][0:1, :]
    ae = _gelu(y0e * scale0 + shift0) + tv
    ao = _gelu(y0o * scale0 + shift0) + tv
    dn = (((1,), (1,)), ((), ()))
    y1e = lax.dot_general(ae, cw1_ref[...], dn,
                          preferred_element_type=jnp.float32) + cb1_ref[...]
    y1o = lax.dot_general(ao, cw1_ref[...], dn,
                          preferred_element_type=jnp.float32) + cb1_ref[...]
    y1b = jnp.concatenate([y1e, y1o], axis=1).astype(jnp.bfloat16)
    y1_ref[0] = y1b
    y1s = y1b.astype(jnp.float32)
    y1a = y1s[:, 0:C1]
    y1c = y1s[:, C1:2 * C1]

    @pl.when((pl.program_id(0) == 0) & (pl.program_id(1) == 0))
    def _():
        st1_ref[...] = jnp.zeros_like(st1_ref)

    st1_ref[...] += jnp.concatenate(
        [(jnp.sum(y1a, axis=0) + jnp.sum(y1c, axis=0))[None, :],
         (jnp.sum(y1a * y1a, axis=0) + jnp.sum(y1c * y1c, axis=0))[None, :]],
        axis=0)


def _p3_body(y1_ref, st1_ref, tv2_ref, cw2_ref, cb2_ref,
             g1_ref, b1_ref, y2_ref, st2_ref):
    scale1, shift1 = _bn_coefs(st1_ref, g1_ref[...], b1_ref[...])
    y1 = y1_ref[0].astype(jnp.float32)
    tv = tv2_ref[0][0:1, :]
    ae = _gelu(y1[:, 0:C1] * scale1 + shift1) + tv
    ao = _gelu(y1[:, C1:2 * C1] * scale1 + shift1) + tv
    dn = (((1,), (1,)), ((), ()))
    y2e = lax.dot_general(ae, cw2_ref[...], dn,
                          preferred_element_type=jnp.float32) + cb2_ref[...]
    y2o = lax.dot_general(ao, cw2_ref[...], dn,
                          preferred_element_type=jnp.float32) + cb2_ref[...]
    y2b = jnp.concatenate([y2e, y2o], axis=1).astype(jnp.bfloat16)
    y2_ref[0] = y2b
    y2s = y2b.astype(jnp.float32)
    y2a = y2s[:, 0:C2]
    y2c = y2s[:, C2:2 * C2]

    @pl.when((pl.program_id(0) == 0) & (pl.program_id(1) == 0))
    def _():
        st2_ref[...] = jnp.zeros_like(st2_ref)

    st2_ref[...] += jnp.concatenate(
        [(jnp.sum(y2a, axis=0) + jnp.sum(y2c, axis=0))[None, :],
         (jnp.sum(y2a * y2a, axis=0) + jnp.sum(y2c * y2c, axis=0))[None, :]],
        axis=0)


def _p4_body(y2_ref, st2_ref, g2_ref, b2_ref, out_ref):
    scale2, shift2 = _bn_coefs(st2_ref, g2_ref[...], b2_ref[...])
    y2 = y2_ref[0].astype(jnp.float32)
    ze = (y2[:, 0:C2] * scale2 + shift2).reshape(RT // KS, KS // 2, C2)
    zo = (y2[:, C2:2 * C2] * scale2 + shift2).reshape(RT // KS, KS // 2, C2)
    zmx = jnp.maximum(jnp.max(ze, axis=1), jnp.max(zo, axis=1))
    zmn = jnp.minimum(jnp.min(ze, axis=1), jnp.min(zo, axis=1))
    m = jnp.maximum(_gelu(zmx), _gelu(zmn))
    out_ref[0] = m.T


def kernel(xyz, points, t_embed, tw0, tb0, cw0, cb0, g0, b0,
           tw1, tb1, cw1, cb1, g1, b1, tw2, tb2, cw2, cb2, g2, b2):
    f32 = jnp.float32
    ptsT = jnp.transpose(points, (0, 2, 1))
    xyzT = jnp.transpose(xyz, (0, 2, 1))
    temb3 = t_embed[:, None, :]

    g_tab, rp, tv1, tv2 = pl.pallas_call(
        _k0_body,
        grid=(BN,),
        in_specs=[
            pl.BlockSpec((1, NN, 3), lambda b: (b, 0, 0)),
            pl.BlockSpec((1, NN, 128), lambda b: (b, 0, 0)),
            pl.BlockSpec((1, 1, 256), lambda b: (b, 0, 0)),
            pl.BlockSpec((131, 256), lambda b: (0, 0)),
            pl.BlockSpec((131,), lambda b: (0,)),
            pl.BlockSpec((64, 131), lambda b: (0, 0)),
            pl.BlockSpec((64,), lambda b: (0,)),
            pl.BlockSpec((64, 256), lambda b: (0, 0)),
            pl.BlockSpec((64,), lambda b: (0,)),
            pl.BlockSpec((128, 256), lambda b: (0, 0)),
            pl.BlockSpec((128,), lambda b: (0,)),
        ],
        out_specs=[
            pl.BlockSpec((1, NN, 2 * C0), lambda b: (b, 0, 0)),
            pl.BlockSpec((1, NN, C0), lambda b: (b, 0, 0)),
            pl.BlockSpec((1, 8, C0), lambda b: (b, 0, 0)),
            pl.BlockSpec((1, 8, C1), lambda b: (b, 0, 0)),
        ],
        out_shape=[
            jax.ShapeDtypeStruct((BN, NN, 2 * C0), f32),
            jax.ShapeDtypeStruct((BN, NN, C0), f32),
            jax.ShapeDtypeStruct((BN, 8, C0), f32),
            jax.ShapeDtypeStruct((BN, 8, C1), f32),
        ],
    )(xyz, ptsT, temb3, tw0, tb0, cw0, cb0, tw1, tb1, tw2, tb2)

    rankp = pl.pallas_call(
        _kmask_body,
        grid=(BN, NN // TS),
        in_specs=[
            pl.BlockSpec((1, TS, 3), lambda b, j: (b, j, 0)),
            pl.BlockSpec((1, 3, NN), lambda b, j: (b, 0, 0)),
        ],
        out_specs=pl.BlockSpec((TS * 16, 128),
                                lambda b, j: (b * (NN // TS) + j, 0)),
        out_shape=jax.ShapeDtypeStruct((BN * NN * 16, 128), jnp.int32),
    )(xyz, xyzT)

    ggath_flat = _sc_call(rankp, g_tab.reshape(BN * NN, 2 * C0))
    ggath = ggath_flat.reshape(BN, NN * KS // 2, 2 * C0)

    st0 = pl.pallas_call(
        _p1_body,
        grid=(BN, NT),
        in_specs=[
            pl.BlockSpec((1, RT // 2, 2 * C0), lambda b, t: (b, t, 0)),
            pl.BlockSpec((1, RT // KS, C0), lambda b, t: (b, t, 0)),
        ],
        out_specs=pl.BlockSpec((2, C0), lambda b, t: (0, 0)),
        out_shape=jax.ShapeDtypeStruct((2, C0), f32),
    )(ggath, rp)

    y1, st1 = pl.pallas_call(
        _p2_body,
        grid=(BN, NT),
        in_specs=[
            pl.BlockSpec((1, RT // 2, 2 * C0), lambda b, t: (b, t, 0)),
            pl.BlockSpec((1, RT // KS, C0), lambda b, t: (b, t, 0)),
            pl.BlockSpec((2, C0), lambda b, t: (0, 0)),
            pl.BlockSpec((1, 8, C0), lambda b, t: (b, 0, 0)),
            pl.BlockSpec((C1, C0), lambda b, t: (0, 0)),
            pl.BlockSpec((C1,), lambda b, t: (0,)),
            pl.BlockSpec((C0,), lambda b, t: (0,)),
            pl.BlockSpec((C0,), lambda b, t: (0,)),
        ],
        out_specs=[
            pl.BlockSpec((1, RT // 2, 2 * C1), lambda b, t: (b, t, 0)),
            pl.BlockSpec((2, C1), lambda b, t: (0, 0)),
        ],
        out_shape=[
            jax.ShapeDtypeStruct((BN, NN * KS // 2, 2 * C1), jnp.bfloat16),
            jax.ShapeDtypeStruct((2, C1), f32),
        ],
    )(ggath, rp, st0, tv1, cw1, cb1, g0, b0)

    y2, st2 = pl.pallas_call(
        _p3_body,
        grid=(BN, NT),
        in_specs=[
            pl.BlockSpec((1, RT // 2, 2 * C1), lambda b, t: (b, t, 0)),
            pl.BlockSpec((2, C1), lambda b, t: (0, 0)),
            pl.BlockSpec((1, 8, C1), lambda b, t: (b, 0, 0)),
            pl.BlockSpec((C2, C1), lambda b, t: (0, 0)),
            pl.BlockSpec((C2,), lambda b, t: (0,)),
            pl.BlockSpec((C1,), lambda b, t: (0,)),
            pl.BlockSpec((C1,), lambda b, t: (0,)),
        ],
        out_specs=[
            pl.BlockSpec((1, RT // 2, 2 * C2), lambda b, t: (b, t, 0)),
            pl.BlockSpec((2, C2), lambda b, t: (0, 0)),
        ],
        out_shape=[
            jax.ShapeDtypeStruct((BN, NN * KS // 2, 2 * C2), jnp.bfloat16),
            jax.ShapeDtypeStruct((2, C2), f32),
        ],
    )(y1, st1, tv2, cw2, cb2, g1, b1)

    out = pl.pallas_call(
        _p4_body,
        grid=(BN, NT),
        in_specs=[
            pl.BlockSpec((1, RT // 2, 2 * C2), lambda b, t: (b, t, 0)),
            pl.BlockSpec((2, C2), lambda b, t: (0, 0)),
            pl.BlockSpec((C2,), lambda b, t: (0,)),
            pl.BlockSpec((C2,), lambda b, t: (0,)),
        ],
        out_specs=pl.BlockSpec((1, C2, RT // KS), lambda b, t: (b, 0, t)),
        out_shape=jax.ShapeDtypeStruct((BN, C2, NN), f32),
    )(y2, st2, g2, b2)

    return out
